# Initial kernel scaffold; baseline (speedup 1.0000x reference)
import jax
import jax.numpy as jnp
from jax.experimental import pallas as pl


def kernel(h, edge_index, coord, edge_attr, We1, be1, We2, be2, Wn1, bn1, Wn2, bn2, Wc1, bc1, Wc2):
    raise NotImplementedError("write your pallas kernel here")


# speedup vs baseline: 2.9286x; 2.9286x over previous
import jax
import jax.numpy as jnp
from jax import lax
from jax.experimental import pallas as pl
from jax.experimental.pallas import tpu as pltpu
from jax.experimental.pallas import tpu_sc as plsc

N = 10000
E = 320000
D = 128
H = 128

NC = 2
NS = 16
NW = NC * NS
EW = E // NW
BE = 80
NCH = EW // BE
LANES = 16

EB = 512
NEB = E // EB

NB = 1000
NNB = N // NB
NB5 = 1024
NNB5 = 10


CROWS = 240


def _cgather(coordv, flat_idx):
    return plsc.load_gather(
        coordv, [lax.shift_right_logical(flat_idx, 7), flat_idx & 127])


def _mesh():
    return plsc.VectorSubcoreMesh(
        core_axis_name="c", subcore_axis_name="s", num_cores=NC, num_subcores=NS
    )


def _k1_body(h_ref, wa_ref, wb_ref, a_ref, b_ref):
    hb = h_ref[...]
    a_ref[...] = jnp.dot(hb, wa_ref[...], preferred_element_type=jnp.float32)
    b_ref[...] = jnp.dot(hb, wb_ref[...], preferred_element_type=jnp.float32)


def _k1(h, wa, wb):
    return pl.pallas_call(
        _k1_body,
        grid=(NNB,),
        in_specs=[
            pl.BlockSpec((NB, D), lambda i: (i, 0)),
            pl.BlockSpec((D, D), lambda i: (0, 0)),
            pl.BlockSpec((D, D), lambda i: (0, 0)),
        ],
        out_specs=[
            pl.BlockSpec((NB, D), lambda i: (i, 0)),
            pl.BlockSpec((NB, D), lambda i: (i, 0)),
        ],
        out_shape=[
            jax.ShapeDtypeStruct((N, D), jnp.float32),
            jax.ShapeDtypeStruct((N, D), jnp.float32),
        ],
    )(h, wa, wb)


def _k2_body(hwa, hwb, rowi, coli, coordf, g_out, rad_out,
             rowv, colv, bufa, bufb, radv, coordv, sema, semb):
    cid = lax.axis_index("c")
    sid = lax.axis_index("s")
    wid = sid * NC + cid
    base0 = wid * EW
    pltpu.sync_copy(coordf, coordv)

    def chunk(ci, carry):
        base = base0 + ci * BE
        pltpu.sync_copy(rowi.at[pl.ds(base, BE)], rowv)
        pltpu.sync_copy(coli.at[pl.ds(base, BE)], colv)
        cpa = pltpu.async_copy(hwa.at[rowv], bufa, sema)
        cpb = pltpu.async_copy(hwb.at[colv], bufb, semb)
        cpa.wait()
        cpb.wait()

        def add_body(t, c):
            i = t // (D // LANES)
            k = lax.rem(t, D // LANES)
            sl = pl.ds(k * LANES, LANES)
            bufa[i, sl] = bufa[i, sl] + bufb[i, sl]
            return c

        lax.fori_loop(0, BE * (D // LANES), add_body, 0)

        def rad_body(g, c):
            sl = pl.ds(g * LANES, LANES)
            r3 = rowv[sl] * 3
            c3 = colv[sl] * 3
            dx = _cgather(coordv, r3) - _cgather(coordv, c3)
            dy = _cgather(coordv, r3 + 1) - _cgather(coordv, c3 + 1)
            dz = _cgather(coordv, r3 + 2) - _cgather(coordv, c3 + 2)
            radv[sl] = dx * dx + dy * dy + dz * dz
            return c

        lax.fori_loop(0, BE // LANES, rad_body, 0)
        pltpu.sync_copy(bufa, g_out.at[pl.ds(base, BE)])
        pltpu.sync_copy(radv, rad_out.at[pl.ds(base, BE)])
        return carry

    lax.fori_loop(0, NCH, chunk, 0)


def _k2(hwa, hwb, rowi, coli, coordf):
    return pl.kernel(
        _k2_body,
        out_type=[
            jax.ShapeDtypeStruct((E, D), jnp.float32),
            jax.ShapeDtypeStruct((E,), jnp.float32),
        ],
        mesh=_mesh(),
        compiler_params=pltpu.CompilerParams(needs_layout_passes=False),
        scratch_types=[
            pltpu.VMEM((BE,), jnp.int32),
            pltpu.VMEM((BE,), jnp.int32),
            pltpu.VMEM((BE, D), jnp.float32),
            pltpu.VMEM((BE, D), jnp.float32),
            pltpu.VMEM((BE,), jnp.float32),
            pltpu.VMEM((CROWS, 128), jnp.float32),
            pltpu.SemaphoreType.DMA,
            pltpu.SemaphoreType.DMA,
        ],
    )(hwa, hwb, rowi, coli, coordf)


def _k3_body(g_ref, ea_ref, rad_ref, wc_ref, wr_ref, be1_ref, w2_ref, be2_ref,
             wc1_ref, bc1_ref, wc2_ref, ef_ref, m_ref):
    rad_row = rad_ref[0]
    radp = lax.dot_general(rad_row, wr_ref[...], (((0,), (0,)), ((), ())),
                           preferred_element_type=jnp.float32)
    pre = (g_ref[...]
           + jnp.dot(ea_ref[...], wc_ref[...], preferred_element_type=jnp.float32)
           + radp + be1_ref[...])
    hdn = jnp.maximum(pre, 0.0)
    ef = jnp.maximum(
        jnp.dot(hdn, w2_ref[...], preferred_element_type=jnp.float32) + be2_ref[...], 0.0)
    ef_ref[...] = ef
    t = jnp.maximum(
        jnp.dot(ef, wc1_ref[...], preferred_element_type=jnp.float32) + bc1_ref[...], 0.0)
    m_ref[0] = lax.dot_general(wc2_ref[...], t, (((0,), (1,)), ((), ())),
                               preferred_element_type=jnp.float32)


def _k3(g, ea, rad2, wc, wr, be1, w2, be2, wc1, bc1, wc2):
    return pl.pallas_call(
        _k3_body,
        grid=(NEB,),
        in_specs=[
            pl.BlockSpec((EB, D), lambda i: (i, 0)),
            pl.BlockSpec((EB, H), lambda i: (i, 0)),
            pl.BlockSpec((1, 1, EB), lambda i: (i, 0, 0)),
            pl.BlockSpec((H, H), lambda i: (0, 0)),
            pl.BlockSpec((1, H), lambda i: (0, 0)),
            pl.BlockSpec((1, H), lambda i: (0, 0)),
            pl.BlockSpec((H, H), lambda i: (0, 0)),
            pl.BlockSpec((1, H), lambda i: (0, 0)),
            pl.BlockSpec((H, H), lambda i: (0, 0)),
            pl.BlockSpec((1, H), lambda i: (0, 0)),
            pl.BlockSpec((H, 1), lambda i: (0, 0)),
        ],
        out_specs=[
            pl.BlockSpec((EB, H), lambda i: (i, 0)),
            pl.BlockSpec((1, 1, EB), lambda i: (i, 0, 0)),
        ],
        out_shape=[
            jax.ShapeDtypeStruct((E, H), jnp.float32),
            jax.ShapeDtypeStruct((NEB, 1, EB), jnp.float32),
        ],
    )(g, ea, rad2, wc, wr, be1, w2, be2, wc1, bc1, wc2)


N_PAD = 10240
NROWS_SUB = N_PAD // NS
ZB = 128
NSR = N_PAD // 8


def _k4a_body(ef, rowi, agg_out, rowv, efv, zbuf, aggS):
    cid = lax.axis_index("c")
    sid = lax.axis_index("s")
    wid = sid * NC + cid
    base0 = wid * EW

    zero16 = jnp.zeros((LANES,), jnp.float32)

    def zb_body(t, c):
        i = t // (D // LANES)
        k = lax.rem(t, D // LANES)
        zbuf[i, pl.ds(k * LANES, LANES)] = zero16
        return c

    lax.fori_loop(0, ZB * (D // LANES), zb_body, 0)

    def zc_body(j, c):
        pltpu.sync_copy(zbuf, aggS.at[pl.ds(sid * NROWS_SUB + j * ZB, ZB)])
        return c

    lax.fori_loop(0, NROWS_SUB // ZB, zc_body, 0)
    plsc.subcore_barrier()

    def chunk(ci, carry):
        base = base0 + ci * BE
        pltpu.sync_copy(rowi.at[pl.ds(base, BE)], rowv)
        pltpu.sync_copy(ef.at[pl.ds(base, BE)], efv)
        pltpu.sync_copy(efv, aggS.at[rowv], add=True)
        return carry

    lax.fori_loop(0, NCH, chunk, 0)
    plsc.subcore_barrier()

    def dump(j, c):
        sl = pl.ds(sid * NROWS_SUB + j * ZB, ZB)
        pltpu.sync_copy(aggS.at[sl], agg_out.at[cid].at[sl])
        return c

    lax.fori_loop(0, NROWS_SUB // ZB, dump, 0)


def _k4a(ef, rowi):
    return pl.kernel(
        _k4a_body,
        out_type=jax.ShapeDtypeStruct((NC, N_PAD, H), jnp.float32),
        mesh=_mesh(),
        compiler_params=pltpu.CompilerParams(needs_layout_passes=False),
        scratch_types=[
            pltpu.VMEM((BE,), jnp.int32),
            pltpu.VMEM((BE, H), jnp.float32),
            pltpu.VMEM((ZB, D), jnp.float32),
            pltpu.VMEM_SHARED((N_PAD, H), jnp.float32),
        ],
    )(ef, rowi)


def _k4b_body(mflat, rowi, coli, coordf, sums_out,
              rowv, colv, rowv8, mv, tbuf, zbuf, coordv, sbuf, ebuf, sumS):
    cid = lax.axis_index("c")
    sid = lax.axis_index("s")
    wid = sid * NC + cid
    base0 = wid * EW
    pltpu.sync_copy(coordf, coordv)

    zero16 = jnp.zeros((LANES,), jnp.float32)
    ones16 = jnp.ones((LANES,), jnp.float32)

    def zb_body(t, c):
        i = t // (D // LANES)
        k = lax.rem(t, D // LANES)
        zbuf[i, pl.ds(k * LANES, LANES)] = zero16
        return c

    lax.fori_loop(0, ZB * (D // LANES), zb_body, 0)

    def zt_body(t, c):
        i = t // (D // LANES)
        k = lax.rem(t, D // LANES)
        tbuf[i, pl.ds(k * LANES, LANES)] = zero16
        return c

    lax.fori_loop(0, BE * (D // LANES), zt_body, 0)

    def ze_body(t, c):
        i = t // (D // LANES)
        k = lax.rem(t, D // LANES)
        ebuf[i, pl.ds(k * LANES, LANES)] = zero16
        return c

    lax.fori_loop(0, 64 * (D // LANES), ze_body, 0)

    pltpu.sync_copy(zbuf.at[pl.ds(0, NSR // NS)], sumS.at[pl.ds(sid * (NSR // NS), NSR // NS)])
    plsc.subcore_barrier()

    def chunk(ci, carry):
        base = base0 + ci * BE
        pltpu.sync_copy(rowi.at[pl.ds(base, BE)], rowv)
        pltpu.sync_copy(coli.at[pl.ds(base, BE)], colv)
        pltpu.sync_copy(mflat.at[pl.ds(base, BE)], mv)

        def tb(g, c):
            sl = pl.ds(g * LANES, LANES)
            r = rowv[sl]
            r3 = r * 3
            c3 = colv[sl] * 3
            m16 = mv[sl]
            dx = _cgather(coordv, r3) - _cgather(coordv, c3)
            dy = _cgather(coordv, r3 + 1) - _cgather(coordv, c3 + 1)
            dz = _cgather(coordv, r3 + 2) - _cgather(coordv, c3 + 2)
            tx = jnp.clip(m16 * dx, -100.0, 100.0)
            ty = jnp.clip(m16 * dy, -100.0, 100.0)
            tz = jnp.clip(m16 * dz, -100.0, 100.0)
            rows = lax.iota(jnp.int32, LANES) + g * LANES
            rowv8[sl] = lax.shift_right_logical(r, 3)
            slot = (r & 7) * LANES
            plsc.store_scatter(tbuf, [rows, slot], tx)
            plsc.store_scatter(tbuf, [rows, slot + 1], ty)
            plsc.store_scatter(tbuf, [rows, slot + 2], tz)
            plsc.store_scatter(tbuf, [rows, slot + 3], ones16)
            return c

        lax.fori_loop(0, BE // LANES, tb, 0)
        pltpu.sync_copy(tbuf, sumS.at[rowv8], add=True)

        def tc(g, c):
            sl = pl.ds(g * LANES, LANES)
            r = rowv[sl]
            rows = lax.iota(jnp.int32, LANES) + g * LANES
            slot = (r & 7) * LANES
            plsc.store_scatter(tbuf, [rows, slot], zero16)
            plsc.store_scatter(tbuf, [rows, slot + 1], zero16)
            plsc.store_scatter(tbuf, [rows, slot + 2], zero16)
            plsc.store_scatter(tbuf, [rows, slot + 3], zero16)
            return c

        lax.fori_loop(0, BE // LANES, tc, 0)
        return carry

    lax.fori_loop(0, NCH, chunk, 0)
    plsc.subcore_barrier()

    def dumpb(j, c):
        nb = sid * NROWS_SUB + j * 64
        pltpu.sync_copy(sumS.at[pl.ds(sid * (NSR // NS) + j * 8, 8)], sbuf)

        def ex(g, cc):
            nloc = lax.iota(jnp.int32, LANES) + g * LANES
            srow = lax.shift_right_logical(nloc, 3)
            scol = (nloc & 7) * LANES
            for k in range(4):
                val = plsc.load_gather(sbuf, [srow, scol + k])
                plsc.store_scatter(ebuf, [nloc, jnp.full((LANES,), k, jnp.int32)], val)
            return cc

        lax.fori_loop(0, 64 // LANES, ex, 0)
        pltpu.sync_copy(ebuf, sums_out.at[cid].at[pl.ds(nb, 64)])
        return c

    lax.fori_loop(0, NROWS_SUB // 64, dumpb, 0)


def _k4b(mflat, rowi, coli, coordf):
    return pl.kernel(
        _k4b_body,
        out_type=jax.ShapeDtypeStruct((NC, N_PAD, D), jnp.float32),
        mesh=_mesh(),
        compiler_params=pltpu.CompilerParams(needs_layout_passes=False),
        scratch_types=[
            pltpu.VMEM((BE,), jnp.int32),
            pltpu.VMEM((BE,), jnp.int32),
            pltpu.VMEM((BE,), jnp.int32),
            pltpu.VMEM((BE,), jnp.float32),
            pltpu.VMEM((BE, D), jnp.float32),
            pltpu.VMEM((ZB, D), jnp.float32),
            pltpu.VMEM((CROWS, 128), jnp.float32),
            pltpu.VMEM((8, D), jnp.float32),
            pltpu.VMEM((64, D), jnp.float32),
            pltpu.VMEM_SHARED((NSR, D), jnp.float32),
        ],
    )(mflat, rowi, coli, coordf)


def _k5_body(h_ref, agg_ref, sums_ref, coord_ref, w1h_ref, w1a_ref, bn1_ref,
             wn2_ref, bn2_ref, hout_ref, cout_ref):
    agg = agg_ref[0] + agg_ref[1]
    hb = h_ref[...]
    t = jnp.maximum(
        jnp.dot(hb, w1h_ref[...], preferred_element_type=jnp.float32)
        + jnp.dot(agg, w1a_ref[...], preferred_element_type=jnp.float32)
        + bn1_ref[...], 0.0)
    hout_ref[...] = hb + jnp.dot(t, wn2_ref[...], preferred_element_type=jnp.float32) + bn2_ref[...]
    s = sums_ref[0] + sums_ref[1]
    cnt = jnp.maximum(s[:, 3:4], 1.0)
    cout_ref[...] = coord_ref[...] + s[:, 0:3] / cnt


def _k5(h, aggp, sumsp, coord, w1h, w1a, bn1, wn2, bn2):
    return pl.pallas_call(
        _k5_body,
        grid=(NNB5,),
        in_specs=[
            pl.BlockSpec((NB5, D), lambda i: (i, 0)),
            pl.BlockSpec((NC, NB5, H), lambda i: (0, i, 0)),
            pl.BlockSpec((NC, NB5, D), lambda i: (0, i, 0)),
            pl.BlockSpec((NB5, 3), lambda i: (i, 0)),
            pl.BlockSpec((D, H), lambda i: (0, 0)),
            pl.BlockSpec((H, H), lambda i: (0, 0)),
            pl.BlockSpec((1, H), lambda i: (0, 0)),
            pl.BlockSpec((H, D), lambda i: (0, 0)),
            pl.BlockSpec((1, D), lambda i: (0, 0)),
        ],
        out_specs=[
            pl.BlockSpec((NB5, D), lambda i: (i, 0)),
            pl.BlockSpec((NB5, 3), lambda i: (i, 0)),
        ],
        out_shape=[
            jax.ShapeDtypeStruct((N, D), jnp.float32),
            jax.ShapeDtypeStruct((N, 3), jnp.float32),
        ],
    )(h, aggp, sumsp, coord, w1h, w1a, bn1, wn2, bn2)


def kernel(h, edge_index, coord, edge_attr, We1, be1, We2, be2,
           Wn1, bn1, Wn2, bn2, Wc1, bc1, Wc2):
    row = edge_index[0]
    col = edge_index[1]
    wa = We1[0:D]
    wb = We1[D:2 * D]
    wr = We1[2 * D:2 * D + 1]
    wc = We1[2 * D + 1:]
    coordf = jnp.pad(coord.reshape(-1), (0, CROWS * 128 - 3 * N)).reshape(CROWS, 128)

    hwa, hwb = _k1(h, wa, wb)
    g, rad = _k2(hwa, hwb, row, col, coordf)
    rad2 = rad.reshape(NEB, 1, EB)
    ef, m2 = _k3(g, edge_attr, rad2, wc, wr, be1.reshape(1, H), We2,
                 be2.reshape(1, H), Wc1, bc1.reshape(1, H), Wc2)
    mflat = m2.reshape(-1)
    aggp = _k4a(ef, row)
    sumsp = _k4b(mflat, row, col, coordf)
    hout, cpart = _k5(h, aggp, sumsp, coord, Wn1[:D], Wn1[D:],
                      bn1.reshape(1, H), Wn2, bn2.reshape(1, D))
    return hout, cpart[:, :, None]


# speedup vs baseline: 2.9431x; 1.0049x over previous
import jax
import jax.numpy as jnp
from jax import lax
from jax.experimental import pallas as pl
from jax.experimental.pallas import tpu as pltpu
from jax.experimental.pallas import tpu_sc as plsc

N = 10000
E = 320000
D = 128
H = 128

NC = 2
NS = 16
NW = NC * NS
EW = E // NW
BE = 80
NCH = EW // BE
LANES = 16

EB = 512
NEB = E // EB

NB = 1000
NNB = N // NB
NB5 = 1024
NNB5 = 10


CROWS = 240


def _bdot(a, b):
    return jnp.dot(a.astype(jnp.bfloat16), b.astype(jnp.bfloat16),
                   preferred_element_type=jnp.float32)


def _cgather(coordv, flat_idx):
    return plsc.load_gather(
        coordv, [lax.shift_right_logical(flat_idx, 7), flat_idx & 127])


def _mesh():
    return plsc.VectorSubcoreMesh(
        core_axis_name="c", subcore_axis_name="s", num_cores=NC, num_subcores=NS
    )


def _k1_body(h_ref, wa_ref, wb_ref, a_ref, b_ref):
    hb = h_ref[...]
    a_ref[...] = _bdot(hb, wa_ref[...])
    b_ref[...] = _bdot(hb, wb_ref[...])


def _k1(h, wa, wb):
    return pl.pallas_call(
        _k1_body,
        grid=(NNB,),
        in_specs=[
            pl.BlockSpec((NB, D), lambda i: (i, 0)),
            pl.BlockSpec((D, D), lambda i: (0, 0)),
            pl.BlockSpec((D, D), lambda i: (0, 0)),
        ],
        out_specs=[
            pl.BlockSpec((NB, D), lambda i: (i, 0)),
            pl.BlockSpec((NB, D), lambda i: (i, 0)),
        ],
        out_shape=[
            jax.ShapeDtypeStruct((N, D), jnp.float32),
            jax.ShapeDtypeStruct((N, D), jnp.float32),
        ],
    )(h, wa, wb)


def _k2_body(hwa, hwb, rowi, coli, coordf, g_out, rad_out,
             rowv, colv, bufa, bufb, radv, coordv, sema, semb):
    cid = lax.axis_index("c")
    sid = lax.axis_index("s")
    wid = sid * NC + cid
    base0 = wid * EW
    pltpu.sync_copy(coordf, coordv)

    def chunk(ci, carry):
        base = base0 + ci * BE
        pltpu.sync_copy(rowi.at[pl.ds(base, BE)], rowv)
        pltpu.sync_copy(coli.at[pl.ds(base, BE)], colv)
        cpa = pltpu.async_copy(hwa.at[rowv], bufa, sema)
        cpb = pltpu.async_copy(hwb.at[colv], bufb, semb)
        cpa.wait()
        cpb.wait()

        def add_body(t, c):
            i = t // (D // LANES)
            k = lax.rem(t, D // LANES)
            sl = pl.ds(k * LANES, LANES)
            bufa[i, sl] = bufa[i, sl] + bufb[i, sl]
            return c

        lax.fori_loop(0, BE * (D // LANES), add_body, 0)

        def rad_body(g, c):
            sl = pl.ds(g * LANES, LANES)
            r3 = rowv[sl] * 3
            c3 = colv[sl] * 3
            dx = _cgather(coordv, r3) - _cgather(coordv, c3)
            dy = _cgather(coordv, r3 + 1) - _cgather(coordv, c3 + 1)
            dz = _cgather(coordv, r3 + 2) - _cgather(coordv, c3 + 2)
            radv[sl] = dx * dx + dy * dy + dz * dz
            return c

        lax.fori_loop(0, BE // LANES, rad_body, 0)
        pltpu.sync_copy(bufa, g_out.at[pl.ds(base, BE)])
        pltpu.sync_copy(radv, rad_out.at[pl.ds(base, BE)])
        return carry

    lax.fori_loop(0, NCH, chunk, 0)


def _k2(hwa, hwb, rowi, coli, coordf):
    return pl.kernel(
        _k2_body,
        out_type=[
            jax.ShapeDtypeStruct((E, D), jnp.float32),
            jax.ShapeDtypeStruct((E,), jnp.float32),
        ],
        mesh=_mesh(),
        compiler_params=pltpu.CompilerParams(needs_layout_passes=False),
        scratch_types=[
            pltpu.VMEM((BE,), jnp.int32),
            pltpu.VMEM((BE,), jnp.int32),
            pltpu.VMEM((BE, D), jnp.float32),
            pltpu.VMEM((BE, D), jnp.float32),
            pltpu.VMEM((BE,), jnp.float32),
            pltpu.VMEM((CROWS, 128), jnp.float32),
            pltpu.SemaphoreType.DMA,
            pltpu.SemaphoreType.DMA,
        ],
    )(hwa, hwb, rowi, coli, coordf)


def _k3_body(g_ref, ea_ref, rad_ref, wc_ref, wr_ref, be1_ref, w2_ref, be2_ref,
             wc1_ref, bc1_ref, wc2_ref, ef_ref, m_ref):
    rad_row = rad_ref[0]
    radp = lax.dot_general(rad_row, wr_ref[...], (((0,), (0,)), ((), ())),
                           preferred_element_type=jnp.float32)
    pre = (g_ref[...] + _bdot(ea_ref[...], wc_ref[...]) + radp + be1_ref[...])
    hdn = jnp.maximum(pre, 0.0)
    ef = jnp.maximum(_bdot(hdn, w2_ref[...]) + be2_ref[...], 0.0)
    ef_ref[...] = ef
    t = jnp.maximum(_bdot(ef, wc1_ref[...]) + bc1_ref[...], 0.0)
    m_ref[0] = lax.dot_general(wc2_ref[...], t, (((0,), (1,)), ((), ())),
                               preferred_element_type=jnp.float32)


def _k3(g, ea, rad2, wc, wr, be1, w2, be2, wc1, bc1, wc2):
    return pl.pallas_call(
        _k3_body,
        grid=(NEB,),
        in_specs=[
            pl.BlockSpec((EB, D), lambda i: (i, 0)),
            pl.BlockSpec((EB, H), lambda i: (i, 0)),
            pl.BlockSpec((1, 1, EB), lambda i: (i, 0, 0)),
            pl.BlockSpec((H, H), lambda i: (0, 0)),
            pl.BlockSpec((1, H), lambda i: (0, 0)),
            pl.BlockSpec((1, H), lambda i: (0, 0)),
            pl.BlockSpec((H, H), lambda i: (0, 0)),
            pl.BlockSpec((1, H), lambda i: (0, 0)),
            pl.BlockSpec((H, H), lambda i: (0, 0)),
            pl.BlockSpec((1, H), lambda i: (0, 0)),
            pl.BlockSpec((H, 1), lambda i: (0, 0)),
        ],
        out_specs=[
            pl.BlockSpec((EB, H), lambda i: (i, 0)),
            pl.BlockSpec((1, 1, EB), lambda i: (i, 0, 0)),
        ],
        out_shape=[
            jax.ShapeDtypeStruct((E, H), jnp.float32),
            jax.ShapeDtypeStruct((NEB, 1, EB), jnp.float32),
        ],
    )(g, ea, rad2, wc, wr, be1, w2, be2, wc1, bc1, wc2)


N_PAD = 10240
NROWS_SUB = N_PAD // NS
ZB = 128
NSR = N_PAD // 8


def _k4a_body(ef, rowi, agg_out, rowv, efv, zbuf, aggS):
    cid = lax.axis_index("c")
    sid = lax.axis_index("s")
    wid = sid * NC + cid
    base0 = wid * EW

    zero16 = jnp.zeros((LANES,), jnp.float32)

    def zb_body(t, c):
        i = t // (D // LANES)
        k = lax.rem(t, D // LANES)
        zbuf[i, pl.ds(k * LANES, LANES)] = zero16
        return c

    lax.fori_loop(0, ZB * (D // LANES), zb_body, 0)

    def zc_body(j, c):
        pltpu.sync_copy(zbuf, aggS.at[pl.ds(sid * NROWS_SUB + j * ZB, ZB)])
        return c

    lax.fori_loop(0, NROWS_SUB // ZB, zc_body, 0)
    plsc.subcore_barrier()

    def chunk(ci, carry):
        base = base0 + ci * BE
        pltpu.sync_copy(rowi.at[pl.ds(base, BE)], rowv)
        pltpu.sync_copy(ef.at[pl.ds(base, BE)], efv)
        pltpu.sync_copy(efv, aggS.at[rowv], add=True)
        return carry

    lax.fori_loop(0, NCH, chunk, 0)
    plsc.subcore_barrier()

    def dump(j, c):
        sl = pl.ds(sid * NROWS_SUB + j * ZB, ZB)
        pltpu.sync_copy(aggS.at[sl], agg_out.at[cid].at[sl])
        return c

    lax.fori_loop(0, NROWS_SUB // ZB, dump, 0)


def _k4a(ef, rowi):
    return pl.kernel(
        _k4a_body,
        out_type=jax.ShapeDtypeStruct((NC, N_PAD, H), jnp.float32),
        mesh=_mesh(),
        compiler_params=pltpu.CompilerParams(needs_layout_passes=False),
        scratch_types=[
            pltpu.VMEM((BE,), jnp.int32),
            pltpu.VMEM((BE, H), jnp.float32),
            pltpu.VMEM((ZB, D), jnp.float32),
            pltpu.VMEM_SHARED((N_PAD, H), jnp.float32),
        ],
    )(ef, rowi)


def _k4b_body(mflat, rowi, coli, coordf, sums_out,
              rowv, colv, rowv8, mv, tbuf, zbuf, coordv, sbuf, ebuf, sumS):
    cid = lax.axis_index("c")
    sid = lax.axis_index("s")
    wid = sid * NC + cid
    base0 = wid * EW
    pltpu.sync_copy(coordf, coordv)

    zero16 = jnp.zeros((LANES,), jnp.float32)
    ones16 = jnp.ones((LANES,), jnp.float32)

    def zb_body(t, c):
        i = t // (D // LANES)
        k = lax.rem(t, D // LANES)
        zbuf[i, pl.ds(k * LANES, LANES)] = zero16
        return c

    lax.fori_loop(0, ZB * (D // LANES), zb_body, 0)

    def zt_body(t, c):
        i = t // (D // LANES)
        k = lax.rem(t, D // LANES)
        tbuf[i, pl.ds(k * LANES, LANES)] = zero16
        return c

    lax.fori_loop(0, BE * (D // LANES), zt_body, 0)

    def ze_body(t, c):
        i = t // (D // LANES)
        k = lax.rem(t, D // LANES)
        ebuf[i, pl.ds(k * LANES, LANES)] = zero16
        return c

    lax.fori_loop(0, 64 * (D // LANES), ze_body, 0)

    pltpu.sync_copy(zbuf.at[pl.ds(0, NSR // NS)], sumS.at[pl.ds(sid * (NSR // NS), NSR // NS)])
    plsc.subcore_barrier()

    def chunk(ci, carry):
        base = base0 + ci * BE
        pltpu.sync_copy(rowi.at[pl.ds(base, BE)], rowv)
        pltpu.sync_copy(coli.at[pl.ds(base, BE)], colv)
        pltpu.sync_copy(mflat.at[pl.ds(base, BE)], mv)

        def tb(g, c):
            sl = pl.ds(g * LANES, LANES)
            r = rowv[sl]
            r3 = r * 3
            c3 = colv[sl] * 3
            m16 = mv[sl]
            dx = _cgather(coordv, r3) - _cgather(coordv, c3)
            dy = _cgather(coordv, r3 + 1) - _cgather(coordv, c3 + 1)
            dz = _cgather(coordv, r3 + 2) - _cgather(coordv, c3 + 2)
            tx = jnp.clip(m16 * dx, -100.0, 100.0)
            ty = jnp.clip(m16 * dy, -100.0, 100.0)
            tz = jnp.clip(m16 * dz, -100.0, 100.0)
            rows = lax.iota(jnp.int32, LANES) + g * LANES
            rowv8[sl] = lax.shift_right_logical(r, 3)
            slot = (r & 7) * LANES
            plsc.store_scatter(tbuf, [rows, slot], tx)
            plsc.store_scatter(tbuf, [rows, slot + 1], ty)
            plsc.store_scatter(tbuf, [rows, slot + 2], tz)
            plsc.store_scatter(tbuf, [rows, slot + 3], ones16)
            return c

        lax.fori_loop(0, BE // LANES, tb, 0)
        pltpu.sync_copy(tbuf, sumS.at[rowv8], add=True)

        def tc(g, c):
            sl = pl.ds(g * LANES, LANES)
            r = rowv[sl]
            rows = lax.iota(jnp.int32, LANES) + g * LANES
            slot = (r & 7) * LANES
            plsc.store_scatter(tbuf, [rows, slot], zero16)
            plsc.store_scatter(tbuf, [rows, slot + 1], zero16)
            plsc.store_scatter(tbuf, [rows, slot + 2], zero16)
            plsc.store_scatter(tbuf, [rows, slot + 3], zero16)
            return c

        lax.fori_loop(0, BE // LANES, tc, 0)
        return carry

    lax.fori_loop(0, NCH, chunk, 0)
    plsc.subcore_barrier()

    def dumpb(j, c):
        nb = sid * NROWS_SUB + j * 64
        pltpu.sync_copy(sumS.at[pl.ds(sid * (NSR // NS) + j * 8, 8)], sbuf)

        def ex(g, cc):
            nloc = lax.iota(jnp.int32, LANES) + g * LANES
            srow = lax.shift_right_logical(nloc, 3)
            scol = (nloc & 7) * LANES
            for k in range(4):
                val = plsc.load_gather(sbuf, [srow, scol + k])
                plsc.store_scatter(ebuf, [nloc, jnp.full((LANES,), k, jnp.int32)], val)
            return cc

        lax.fori_loop(0, 64 // LANES, ex, 0)
        pltpu.sync_copy(ebuf, sums_out.at[cid].at[pl.ds(nb, 64)])
        return c

    lax.fori_loop(0, NROWS_SUB // 64, dumpb, 0)


def _k4b(mflat, rowi, coli, coordf):
    return pl.kernel(
        _k4b_body,
        out_type=jax.ShapeDtypeStruct((NC, N_PAD, D), jnp.float32),
        mesh=_mesh(),
        compiler_params=pltpu.CompilerParams(needs_layout_passes=False),
        scratch_types=[
            pltpu.VMEM((BE,), jnp.int32),
            pltpu.VMEM((BE,), jnp.int32),
            pltpu.VMEM((BE,), jnp.int32),
            pltpu.VMEM((BE,), jnp.float32),
            pltpu.VMEM((BE, D), jnp.float32),
            pltpu.VMEM((ZB, D), jnp.float32),
            pltpu.VMEM((CROWS, 128), jnp.float32),
            pltpu.VMEM((8, D), jnp.float32),
            pltpu.VMEM((64, D), jnp.float32),
            pltpu.VMEM_SHARED((NSR, D), jnp.float32),
        ],
    )(mflat, rowi, coli, coordf)


def _k5_body(h_ref, agg_ref, sums_ref, coord_ref, w1h_ref, w1a_ref, bn1_ref,
             wn2_ref, bn2_ref, hout_ref, cout_ref):
    agg = agg_ref[0] + agg_ref[1]
    hb = h_ref[...]
    t = jnp.maximum(
        _bdot(hb, w1h_ref[...]) + _bdot(agg, w1a_ref[...]) + bn1_ref[...], 0.0)
    hout_ref[...] = hb + _bdot(t, wn2_ref[...]) + bn2_ref[...]
    s = sums_ref[0] + sums_ref[1]
    cnt = jnp.maximum(s[:, 3:4], 1.0)
    cout_ref[...] = coord_ref[...] + s[:, 0:3] / cnt


def _k5(h, aggp, sumsp, coord, w1h, w1a, bn1, wn2, bn2):
    return pl.pallas_call(
        _k5_body,
        grid=(NNB5,),
        in_specs=[
            pl.BlockSpec((NB5, D), lambda i: (i, 0)),
            pl.BlockSpec((NC, NB5, H), lambda i: (0, i, 0)),
            pl.BlockSpec((NC, NB5, D), lambda i: (0, i, 0)),
            pl.BlockSpec((NB5, 3), lambda i: (i, 0)),
            pl.BlockSpec((D, H), lambda i: (0, 0)),
            pl.BlockSpec((H, H), lambda i: (0, 0)),
            pl.BlockSpec((1, H), lambda i: (0, 0)),
            pl.BlockSpec((H, D), lambda i: (0, 0)),
            pl.BlockSpec((1, D), lambda i: (0, 0)),
        ],
        out_specs=[
            pl.BlockSpec((NB5, D), lambda i: (i, 0)),
            pl.BlockSpec((NB5, 3), lambda i: (i, 0)),
        ],
        out_shape=[
            jax.ShapeDtypeStruct((N, D), jnp.float32),
            jax.ShapeDtypeStruct((N, 3), jnp.float32),
        ],
    )(h, aggp, sumsp, coord, w1h, w1a, bn1, wn2, bn2)


def kernel(h, edge_index, coord, edge_attr, We1, be1, We2, be2,
           Wn1, bn1, Wn2, bn2, Wc1, bc1, Wc2):
    row = edge_index[0]
    col = edge_index[1]
    wa = We1[0:D]
    wb = We1[D:2 * D]
    wr = We1[2 * D:2 * D + 1]
    wc = We1[2 * D + 1:]
    coordf = jnp.pad(coord.reshape(-1), (0, CROWS * 128 - 3 * N)).reshape(CROWS, 128)

    hwa, hwb = _k1(h, wa, wb)
    g, rad = _k2(hwa, hwb, row, col, coordf)
    rad2 = rad.reshape(NEB, 1, EB)
    ef, m2 = _k3(g, edge_attr, rad2, wc, wr, be1.reshape(1, H), We2,
                 be2.reshape(1, H), Wc1, bc1.reshape(1, H), Wc2)
    mflat = m2.reshape(-1)
    aggp = _k4a(ef, row)
    sumsp = _k4b(mflat, row, col, coordf)
    hout, cpart = _k5(h, aggp, sumsp, coord, Wn1[:D], Wn1[D:],
                      bn1.reshape(1, H), Wn2, bn2.reshape(1, D))
    return hout, cpart[:, :, None]


# speedup vs baseline: 3.5574x; 1.2087x over previous
import jax
import jax.numpy as jnp
from jax import lax
from jax.experimental import pallas as pl
from jax.experimental.pallas import tpu as pltpu
from jax.experimental.pallas import tpu_sc as plsc

N = 10000
E = 320000
D = 128
H = 128

NC = 2
NS = 16
NW = NC * NS
EW = E // NW
BE = 80
NCH = EW // BE
LANES = 16

EB = 2000
NEB = E // EB

NB = 1000
NNB = N // NB
NB5 = 1024
NNB5 = 10


CROWS = 240


def _bdot(a, b):
    return jnp.dot(a.astype(jnp.bfloat16), b.astype(jnp.bfloat16),
                   preferred_element_type=jnp.float32)


def _cgather(coordv, flat_idx):
    return plsc.load_gather(
        coordv, [lax.shift_right_logical(flat_idx, 7), flat_idx & 127])


def _mesh():
    return plsc.VectorSubcoreMesh(
        core_axis_name="c", subcore_axis_name="s", num_cores=NC, num_subcores=NS
    )


def _k1_body(h_ref, wa_ref, wb_ref, a_ref, b_ref):
    hb = h_ref[...]
    a_ref[...] = _bdot(hb, wa_ref[...])
    b_ref[...] = _bdot(hb, wb_ref[...])


def _k1(h, wa, wb):
    return pl.pallas_call(
        _k1_body,
        grid=(NNB,),
        in_specs=[
            pl.BlockSpec((NB, D), lambda i: (i, 0)),
            pl.BlockSpec((D, D), lambda i: (0, 0)),
            pl.BlockSpec((D, D), lambda i: (0, 0)),
        ],
        out_specs=[
            pl.BlockSpec((NB, D), lambda i: (i, 0)),
            pl.BlockSpec((NB, D), lambda i: (i, 0)),
        ],
        out_shape=[
            jax.ShapeDtypeStruct((N, D), jnp.float32),
            jax.ShapeDtypeStruct((N, D), jnp.float32),
        ],
    )(h, wa, wb)


def _k2_body(hwa, hwb, rowi, coli, coordf, g_out, rad_out,
             rowv, colv, bufa, bufb, radv, coordv, sema, semb):
    cid = lax.axis_index("c")
    sid = lax.axis_index("s")
    wid = sid * NC + cid
    base0 = wid * EW
    pltpu.sync_copy(coordf, coordv)

    def chunk(ci, carry):
        base = base0 + ci * BE
        pltpu.sync_copy(rowi.at[pl.ds(base, BE)], rowv)
        pltpu.sync_copy(coli.at[pl.ds(base, BE)], colv)
        cpa = pltpu.async_copy(hwa.at[rowv], bufa, sema)
        cpb = pltpu.async_copy(hwb.at[colv], bufb, semb)
        cpa.wait()
        cpb.wait()

        def add_body(t, c):
            i = t // (D // LANES)
            k = lax.rem(t, D // LANES)
            sl = pl.ds(k * LANES, LANES)
            bufa[i, sl] = bufa[i, sl] + bufb[i, sl]
            return c

        lax.fori_loop(0, BE * (D // LANES), add_body, 0)

        def rad_body(g, c):
            sl = pl.ds(g * LANES, LANES)
            r3 = rowv[sl] * 3
            c3 = colv[sl] * 3
            dx = _cgather(coordv, r3) - _cgather(coordv, c3)
            dy = _cgather(coordv, r3 + 1) - _cgather(coordv, c3 + 1)
            dz = _cgather(coordv, r3 + 2) - _cgather(coordv, c3 + 2)
            radv[sl] = dx * dx + dy * dy + dz * dz
            return c

        lax.fori_loop(0, BE // LANES, rad_body, 0)
        pltpu.sync_copy(bufa, g_out.at[pl.ds(base, BE)])
        pltpu.sync_copy(radv, rad_out.at[pl.ds(base, BE)])
        return carry

    lax.fori_loop(0, NCH, chunk, 0)


def _k2(hwa, hwb, rowi, coli, coordf):
    return pl.kernel(
        _k2_body,
        out_type=[
            jax.ShapeDtypeStruct((E, D), jnp.float32),
            jax.ShapeDtypeStruct((E,), jnp.float32),
        ],
        mesh=_mesh(),
        compiler_params=pltpu.CompilerParams(needs_layout_passes=False),
        scratch_types=[
            pltpu.VMEM((BE,), jnp.int32),
            pltpu.VMEM((BE,), jnp.int32),
            pltpu.VMEM((BE, D), jnp.float32),
            pltpu.VMEM((BE, D), jnp.float32),
            pltpu.VMEM((BE,), jnp.float32),
            pltpu.VMEM((CROWS, 128), jnp.float32),
            pltpu.SemaphoreType.DMA,
            pltpu.SemaphoreType.DMA,
        ],
    )(hwa, hwb, rowi, coli, coordf)


def _k3_body(g_ref, ea_ref, rad_ref, wc_ref, wr_ref, be1_ref, w2_ref, be2_ref,
             wc1_ref, bc1_ref, wc2_ref, ef_ref, m_ref):
    rad_row = rad_ref[0]
    radp = lax.dot_general(rad_row, wr_ref[...], (((0,), (0,)), ((), ())),
                           preferred_element_type=jnp.float32)
    pre = (g_ref[...] + _bdot(ea_ref[...], wc_ref[...]) + radp + be1_ref[...])
    hdn = jnp.maximum(pre, 0.0)
    ef = jnp.maximum(_bdot(hdn, w2_ref[...]) + be2_ref[...], 0.0)
    ef_ref[...] = ef
    t = jnp.maximum(_bdot(ef, wc1_ref[...]) + bc1_ref[...], 0.0)
    m_ref[0] = lax.dot_general(wc2_ref[...], t, (((0,), (1,)), ((), ())),
                               preferred_element_type=jnp.float32)


def _k3(g, ea, rad2, wc, wr, be1, w2, be2, wc1, bc1, wc2):
    return pl.pallas_call(
        _k3_body,
        grid=(NEB,),
        in_specs=[
            pl.BlockSpec((EB, D), lambda i: (i, 0)),
            pl.BlockSpec((EB, H), lambda i: (i, 0)),
            pl.BlockSpec((1, 1, EB), lambda i: (i, 0, 0)),
            pl.BlockSpec((H, H), lambda i: (0, 0)),
            pl.BlockSpec((1, H), lambda i: (0, 0)),
            pl.BlockSpec((1, H), lambda i: (0, 0)),
            pl.BlockSpec((H, H), lambda i: (0, 0)),
            pl.BlockSpec((1, H), lambda i: (0, 0)),
            pl.BlockSpec((H, H), lambda i: (0, 0)),
            pl.BlockSpec((1, H), lambda i: (0, 0)),
            pl.BlockSpec((H, 1), lambda i: (0, 0)),
        ],
        out_specs=[
            pl.BlockSpec((EB, H), lambda i: (i, 0)),
            pl.BlockSpec((1, 1, EB), lambda i: (i, 0, 0)),
        ],
        out_shape=[
            jax.ShapeDtypeStruct((E, H), jnp.float32),
            jax.ShapeDtypeStruct((NEB, 1, EB), jnp.float32),
        ],
    )(g, ea, rad2, wc, wr, be1, w2, be2, wc1, bc1, wc2)


N_PAD = 10240
NROWS_SUB = N_PAD // NS
ZB = 128
NSR = N_PAD // 8


def _k4a_body(ef, rowi, agg_out, rowv, efv, zbuf, aggS):
    cid = lax.axis_index("c")
    sid = lax.axis_index("s")
    wid = sid * NC + cid
    base0 = wid * EW

    zero16 = jnp.zeros((LANES,), jnp.float32)

    def zb_body(t, c):
        i = t // (D // LANES)
        k = lax.rem(t, D // LANES)
        zbuf[i, pl.ds(k * LANES, LANES)] = zero16
        return c

    lax.fori_loop(0, ZB * (D // LANES), zb_body, 0)

    def zc_body(j, c):
        pltpu.sync_copy(zbuf, aggS.at[pl.ds(sid * NROWS_SUB + j * ZB, ZB)])
        return c

    lax.fori_loop(0, NROWS_SUB // ZB, zc_body, 0)
    plsc.subcore_barrier()

    def chunk(ci, carry):
        base = base0 + ci * BE
        pltpu.sync_copy(rowi.at[pl.ds(base, BE)], rowv)
        pltpu.sync_copy(ef.at[pl.ds(base, BE)], efv)
        pltpu.sync_copy(efv, aggS.at[rowv], add=True)
        return carry

    lax.fori_loop(0, NCH, chunk, 0)
    plsc.subcore_barrier()

    def dump(j, c):
        sl = pl.ds(sid * NROWS_SUB + j * ZB, ZB)
        pltpu.sync_copy(aggS.at[sl], agg_out.at[cid].at[sl])
        return c

    lax.fori_loop(0, NROWS_SUB // ZB, dump, 0)


def _k4a(ef, rowi):
    return pl.kernel(
        _k4a_body,
        out_type=jax.ShapeDtypeStruct((NC, N_PAD, H), jnp.float32),
        mesh=_mesh(),
        compiler_params=pltpu.CompilerParams(needs_layout_passes=False),
        scratch_types=[
            pltpu.VMEM((BE,), jnp.int32),
            pltpu.VMEM((BE, H), jnp.float32),
            pltpu.VMEM((ZB, D), jnp.float32),
            pltpu.VMEM_SHARED((N_PAD, H), jnp.float32),
        ],
    )(ef, rowi)


def _k4b_body(mflat, rowi, coli, coordf, sums_out,
              rowv, colv, rowv8, mv, tbuf, zbuf, coordv, sbuf, ebuf, sumS):
    cid = lax.axis_index("c")
    sid = lax.axis_index("s")
    wid = sid * NC + cid
    base0 = wid * EW
    pltpu.sync_copy(coordf, coordv)

    zero16 = jnp.zeros((LANES,), jnp.float32)
    ones16 = jnp.ones((LANES,), jnp.float32)

    def zb_body(t, c):
        i = t // (D // LANES)
        k = lax.rem(t, D // LANES)
        zbuf[i, pl.ds(k * LANES, LANES)] = zero16
        return c

    lax.fori_loop(0, ZB * (D // LANES), zb_body, 0)

    def zt_body(t, c):
        i = t // (D // LANES)
        k = lax.rem(t, D // LANES)
        tbuf[i, pl.ds(k * LANES, LANES)] = zero16
        return c

    lax.fori_loop(0, BE * (D // LANES), zt_body, 0)

    def ze_body(t, c):
        i = t // (D // LANES)
        k = lax.rem(t, D // LANES)
        ebuf[i, pl.ds(k * LANES, LANES)] = zero16
        return c

    lax.fori_loop(0, 64 * (D // LANES), ze_body, 0)

    pltpu.sync_copy(zbuf.at[pl.ds(0, NSR // NS)], sumS.at[pl.ds(sid * (NSR // NS), NSR // NS)])
    plsc.subcore_barrier()

    def chunk(ci, carry):
        base = base0 + ci * BE
        pltpu.sync_copy(rowi.at[pl.ds(base, BE)], rowv)
        pltpu.sync_copy(coli.at[pl.ds(base, BE)], colv)
        pltpu.sync_copy(mflat.at[pl.ds(base, BE)], mv)

        def tb(g, c):
            sl = pl.ds(g * LANES, LANES)
            r = rowv[sl]
            r3 = r * 3
            c3 = colv[sl] * 3
            m16 = mv[sl]
            dx = _cgather(coordv, r3) - _cgather(coordv, c3)
            dy = _cgather(coordv, r3 + 1) - _cgather(coordv, c3 + 1)
            dz = _cgather(coordv, r3 + 2) - _cgather(coordv, c3 + 2)
            tx = jnp.clip(m16 * dx, -100.0, 100.0)
            ty = jnp.clip(m16 * dy, -100.0, 100.0)
            tz = jnp.clip(m16 * dz, -100.0, 100.0)
            rows = lax.iota(jnp.int32, LANES) + g * LANES
            rowv8[sl] = lax.shift_right_logical(r, 3)
            slot = (r & 7) * LANES
            plsc.store_scatter(tbuf, [rows, slot], tx)
            plsc.store_scatter(tbuf, [rows, slot + 1], ty)
            plsc.store_scatter(tbuf, [rows, slot + 2], tz)
            plsc.store_scatter(tbuf, [rows, slot + 3], ones16)
            return c

        lax.fori_loop(0, BE // LANES, tb, 0)
        pltpu.sync_copy(tbuf, sumS.at[rowv8], add=True)

        def tc(g, c):
            sl = pl.ds(g * LANES, LANES)
            r = rowv[sl]
            rows = lax.iota(jnp.int32, LANES) + g * LANES
            slot = (r & 7) * LANES
            plsc.store_scatter(tbuf, [rows, slot], zero16)
            plsc.store_scatter(tbuf, [rows, slot + 1], zero16)
            plsc.store_scatter(tbuf, [rows, slot + 2], zero16)
            plsc.store_scatter(tbuf, [rows, slot + 3], zero16)
            return c

        lax.fori_loop(0, BE // LANES, tc, 0)
        return carry

    lax.fori_loop(0, NCH, chunk, 0)
    plsc.subcore_barrier()

    def dumpb(j, c):
        nb = sid * NROWS_SUB + j * 64
        pltpu.sync_copy(sumS.at[pl.ds(sid * (NSR // NS) + j * 8, 8)], sbuf)

        def ex(g, cc):
            nloc = lax.iota(jnp.int32, LANES) + g * LANES
            srow = lax.shift_right_logical(nloc, 3)
            scol = (nloc & 7) * LANES
            for k in range(4):
                val = plsc.load_gather(sbuf, [srow, scol + k])
                plsc.store_scatter(ebuf, [nloc, jnp.full((LANES,), k, jnp.int32)], val)
            return cc

        lax.fori_loop(0, 64 // LANES, ex, 0)
        pltpu.sync_copy(ebuf, sums_out.at[cid].at[pl.ds(nb, 64)])
        return c

    lax.fori_loop(0, NROWS_SUB // 64, dumpb, 0)


def _k4b(mflat, rowi, coli, coordf):
    return pl.kernel(
        _k4b_body,
        out_type=jax.ShapeDtypeStruct((NC, N_PAD, D), jnp.float32),
        mesh=_mesh(),
        compiler_params=pltpu.CompilerParams(needs_layout_passes=False),
        scratch_types=[
            pltpu.VMEM((BE,), jnp.int32),
            pltpu.VMEM((BE,), jnp.int32),
            pltpu.VMEM((BE,), jnp.int32),
            pltpu.VMEM((BE,), jnp.float32),
            pltpu.VMEM((BE, D), jnp.float32),
            pltpu.VMEM((ZB, D), jnp.float32),
            pltpu.VMEM((CROWS, 128), jnp.float32),
            pltpu.VMEM((8, D), jnp.float32),
            pltpu.VMEM((64, D), jnp.float32),
            pltpu.VMEM_SHARED((NSR, D), jnp.float32),
        ],
    )(mflat, rowi, coli, coordf)


def _k5_body(h_ref, agg_ref, sums_ref, coord_ref, w1h_ref, w1a_ref, bn1_ref,
             wn2_ref, bn2_ref, hout_ref, cout_ref):
    agg = agg_ref[0] + agg_ref[1]
    hb = h_ref[...]
    t = jnp.maximum(
        _bdot(hb, w1h_ref[...]) + _bdot(agg, w1a_ref[...]) + bn1_ref[...], 0.0)
    hout_ref[...] = hb + _bdot(t, wn2_ref[...]) + bn2_ref[...]
    s = sums_ref[0] + sums_ref[1]
    cnt = jnp.maximum(s[:, 3:4], 1.0)
    cout_ref[...] = coord_ref[...] + s[:, 0:3] / cnt


def _k5(h, aggp, sumsp, coord, w1h, w1a, bn1, wn2, bn2):
    return pl.pallas_call(
        _k5_body,
        grid=(NNB5,),
        in_specs=[
            pl.BlockSpec((NB5, D), lambda i: (i, 0)),
            pl.BlockSpec((NC, NB5, H), lambda i: (0, i, 0)),
            pl.BlockSpec((NC, NB5, D), lambda i: (0, i, 0)),
            pl.BlockSpec((NB5, 3), lambda i: (i, 0)),
            pl.BlockSpec((D, H), lambda i: (0, 0)),
            pl.BlockSpec((H, H), lambda i: (0, 0)),
            pl.BlockSpec((1, H), lambda i: (0, 0)),
            pl.BlockSpec((H, D), lambda i: (0, 0)),
            pl.BlockSpec((1, D), lambda i: (0, 0)),
        ],
        out_specs=[
            pl.BlockSpec((NB5, D), lambda i: (i, 0)),
            pl.BlockSpec((NB5, 3), lambda i: (i, 0)),
        ],
        out_shape=[
            jax.ShapeDtypeStruct((N, D), jnp.float32),
            jax.ShapeDtypeStruct((N, 3), jnp.float32),
        ],
    )(h, aggp, sumsp, coord, w1h, w1a, bn1, wn2, bn2)


def kernel(h, edge_index, coord, edge_attr, We1, be1, We2, be2,
           Wn1, bn1, Wn2, bn2, Wc1, bc1, Wc2):
    row = edge_index[0]
    col = edge_index[1]
    wa = We1[0:D]
    wb = We1[D:2 * D]
    wr = We1[2 * D:2 * D + 1]
    wc = We1[2 * D + 1:]
    coordf = jnp.pad(coord.reshape(-1), (0, CROWS * 128 - 3 * N)).reshape(CROWS, 128)

    hwa, hwb = _k1(h, wa, wb)
    g, rad = _k2(hwa, hwb, row, col, coordf)
    rad2 = rad.reshape(NEB, 1, EB)
    ef, m2 = _k3(g, edge_attr, rad2, wc, wr, be1.reshape(1, H), We2,
                 be2.reshape(1, H), Wc1, bc1.reshape(1, H), Wc2)
    mflat = m2.reshape(-1)
    aggp = _k4a(ef, row)
    sumsp = _k4b(mflat, row, col, coordf)
    hout, cpart = _k5(h, aggp, sumsp, coord, Wn1[:D], Wn1[D:],
                      bn1.reshape(1, H), Wn2, bn2.reshape(1, D))
    return hout, cpart[:, :, None]


# speedup vs baseline: 4.9145x; 1.3815x over previous
import jax
import jax.numpy as jnp
from jax import lax
from jax.experimental import pallas as pl
from jax.experimental.pallas import tpu as pltpu
from jax.experimental.pallas import tpu_sc as plsc

N = 10000
E = 320000
D = 128
H = 128

NC = 2
NS = 16
NW = NC * NS
EW = E // NW
BE = 80
NCH = EW // BE
LANES = 16

EB = 2000
NEB = E // EB

NB = 1000
NNB = N // NB
NB5 = 1024
NNB5 = 10


CROWS = 240


def _bdot(a, b):
    return jnp.dot(a.astype(jnp.bfloat16), b.astype(jnp.bfloat16),
                   preferred_element_type=jnp.float32)


def _cgather(coordv, flat_idx):
    return plsc.load_gather(
        coordv, [lax.shift_right_logical(flat_idx, 7), flat_idx & 127])


def _mesh():
    return plsc.VectorSubcoreMesh(
        core_axis_name="c", subcore_axis_name="s", num_cores=NC, num_subcores=NS
    )


def _k1_body(h_ref, wa_ref, wb_ref, a_ref, b_ref):
    hb = h_ref[...]
    a_ref[...] = _bdot(hb, wa_ref[...])
    b_ref[...] = _bdot(hb, wb_ref[...])


def _k1(h, wa, wb):
    return pl.pallas_call(
        _k1_body,
        grid=(NNB,),
        in_specs=[
            pl.BlockSpec((NB, D), lambda i: (i, 0)),
            pl.BlockSpec((D, D), lambda i: (0, 0)),
            pl.BlockSpec((D, D), lambda i: (0, 0)),
        ],
        out_specs=[
            pl.BlockSpec((NB, D), lambda i: (i, 0)),
            pl.BlockSpec((NB, D), lambda i: (i, 0)),
        ],
        out_shape=[
            jax.ShapeDtypeStruct((N, D), jnp.float32),
            jax.ShapeDtypeStruct((N, D), jnp.float32),
        ],
    )(h, wa, wb)


def _k2_body(hwa, hwb, rowi, coli, coordf, g_out, rad_out,
             rowv0, rowv1, colv0, colv1, bufa0, bufa1, bufb0, bufb1,
             radv, coordv, sema0, sema1, semb0, semb1):
    cid = lax.axis_index("c")
    sid = lax.axis_index("s")
    wid = sid * NC + cid
    base0 = wid * EW
    pltpu.sync_copy(coordf, coordv)

    ROWV = [rowv0, rowv1]
    COLV = [colv0, colv1]
    BUFA = [bufa0, bufa1]
    BUFB = [bufb0, bufb1]
    SEMA = [sema0, sema1]
    SEMB = [semb0, semb1]

    def io(ci, b):
        base = base0 + ci * BE
        pltpu.sync_copy(rowi.at[pl.ds(base, BE)], ROWV[b])
        pltpu.sync_copy(coli.at[pl.ds(base, BE)], COLV[b])
        pltpu.async_copy(hwa.at[ROWV[b]], BUFA[b], SEMA[b])
        pltpu.async_copy(hwb.at[COLV[b]], BUFB[b], SEMB[b])

    def compute(ci, b):
        base = base0 + ci * BE
        pltpu.make_async_copy(hwa.at[ROWV[b]], BUFA[b], SEMA[b]).wait()
        pltpu.make_async_copy(hwb.at[COLV[b]], BUFB[b], SEMB[b]).wait()
        ba, bb = BUFA[b], BUFB[b]

        def add_body(i, c):
            for k in range(D // LANES):
                sl = pl.ds(k * LANES, LANES)
                ba[i, sl] = ba[i, sl] + bb[i, sl]
            return c

        lax.fori_loop(0, BE, add_body, 0)

        rv, cv = ROWV[b], COLV[b]

        def rad_body(g, c):
            sl = pl.ds(g * LANES, LANES)
            r3 = rv[sl] * 3
            c3 = cv[sl] * 3
            dx = _cgather(coordv, r3) - _cgather(coordv, c3)
            dy = _cgather(coordv, r3 + 1) - _cgather(coordv, c3 + 1)
            dz = _cgather(coordv, r3 + 2) - _cgather(coordv, c3 + 2)
            radv[sl] = dx * dx + dy * dy + dz * dz
            return c

        lax.fori_loop(0, BE // LANES, rad_body, 0)
        pltpu.sync_copy(ba, g_out.at[pl.ds(base, BE)])
        pltpu.sync_copy(radv, rad_out.at[pl.ds(base, BE)])

    io(0, 0)

    def pipe(j, carry):
        c0 = j * 2
        io(c0 + 1, 1)
        compute(c0, 0)
        io(c0 + 2, 0)
        compute(c0 + 1, 1)
        return carry

    lax.fori_loop(0, (NCH - 1) // 2, pipe, 0)
    compute(NCH - 1, 0)


def _k2(hwa, hwb, rowi, coli, coordf):
    return pl.kernel(
        _k2_body,
        out_type=[
            jax.ShapeDtypeStruct((E, D), jnp.float32),
            jax.ShapeDtypeStruct((E,), jnp.float32),
        ],
        mesh=_mesh(),
        compiler_params=pltpu.CompilerParams(needs_layout_passes=False),
        scratch_types=[
            pltpu.VMEM((BE,), jnp.int32),
            pltpu.VMEM((BE,), jnp.int32),
            pltpu.VMEM((BE,), jnp.int32),
            pltpu.VMEM((BE,), jnp.int32),
            pltpu.VMEM((BE, D), jnp.float32),
            pltpu.VMEM((BE, D), jnp.float32),
            pltpu.VMEM((BE, D), jnp.float32),
            pltpu.VMEM((BE, D), jnp.float32),
            pltpu.VMEM((BE,), jnp.float32),
            pltpu.VMEM((CROWS, 128), jnp.float32),
            pltpu.SemaphoreType.DMA,
            pltpu.SemaphoreType.DMA,
            pltpu.SemaphoreType.DMA,
            pltpu.SemaphoreType.DMA,
        ],
    )(hwa, hwb, rowi, coli, coordf)


def _k3_body(g_ref, ea_ref, rad_ref, wc_ref, wr_ref, be1_ref, w2_ref, be2_ref,
             wc1_ref, bc1_ref, wc2_ref, ef_ref, m_ref):
    rad_row = rad_ref[0]
    radp = lax.dot_general(rad_row, wr_ref[...], (((0,), (0,)), ((), ())),
                           preferred_element_type=jnp.float32)
    pre = (g_ref[...] + _bdot(ea_ref[...], wc_ref[...]) + radp + be1_ref[...])
    hdn = jnp.maximum(pre, 0.0)
    ef = jnp.maximum(_bdot(hdn, w2_ref[...]) + be2_ref[...], 0.0)
    ef_ref[...] = ef
    t = jnp.maximum(_bdot(ef, wc1_ref[...]) + bc1_ref[...], 0.0)
    m_ref[0] = lax.dot_general(wc2_ref[...], t, (((0,), (1,)), ((), ())),
                               preferred_element_type=jnp.float32)


def _k3(g, ea, rad2, wc, wr, be1, w2, be2, wc1, bc1, wc2):
    return pl.pallas_call(
        _k3_body,
        grid=(NEB,),
        in_specs=[
            pl.BlockSpec((EB, D), lambda i: (i, 0)),
            pl.BlockSpec((EB, H), lambda i: (i, 0)),
            pl.BlockSpec((1, 1, EB), lambda i: (i, 0, 0)),
            pl.BlockSpec((H, H), lambda i: (0, 0)),
            pl.BlockSpec((1, H), lambda i: (0, 0)),
            pl.BlockSpec((1, H), lambda i: (0, 0)),
            pl.BlockSpec((H, H), lambda i: (0, 0)),
            pl.BlockSpec((1, H), lambda i: (0, 0)),
            pl.BlockSpec((H, H), lambda i: (0, 0)),
            pl.BlockSpec((1, H), lambda i: (0, 0)),
            pl.BlockSpec((H, 1), lambda i: (0, 0)),
        ],
        out_specs=[
            pl.BlockSpec((EB, H), lambda i: (i, 0)),
            pl.BlockSpec((1, 1, EB), lambda i: (i, 0, 0)),
        ],
        out_shape=[
            jax.ShapeDtypeStruct((E, H), jnp.float32),
            jax.ShapeDtypeStruct((NEB, 1, EB), jnp.float32),
        ],
    )(g, ea, rad2, wc, wr, be1, w2, be2, wc1, bc1, wc2)


N_PAD = 10240
NROWS_SUB = N_PAD // NS
ZB = 128
NSR = N_PAD // 8


def _k4a_body(ef, rowi, agg_out, rowv, efv, zbuf, aggS):
    cid = lax.axis_index("c")
    sid = lax.axis_index("s")
    wid = sid * NC + cid
    base0 = wid * EW

    zero16 = jnp.zeros((LANES,), jnp.float32)

    def zb_body(t, c):
        i = t // (D // LANES)
        k = lax.rem(t, D // LANES)
        zbuf[i, pl.ds(k * LANES, LANES)] = zero16
        return c

    lax.fori_loop(0, ZB * (D // LANES), zb_body, 0)

    def zc_body(j, c):
        pltpu.sync_copy(zbuf, aggS.at[pl.ds(sid * NROWS_SUB + j * ZB, ZB)])
        return c

    lax.fori_loop(0, NROWS_SUB // ZB, zc_body, 0)
    plsc.subcore_barrier()

    def chunk(ci, carry):
        base = base0 + ci * BE
        pltpu.sync_copy(rowi.at[pl.ds(base, BE)], rowv)
        pltpu.sync_copy(ef.at[pl.ds(base, BE)], efv)
        pltpu.sync_copy(efv, aggS.at[rowv], add=True)
        return carry

    lax.fori_loop(0, NCH, chunk, 0)
    plsc.subcore_barrier()

    def dump(j, c):
        sl = pl.ds(sid * NROWS_SUB + j * ZB, ZB)
        pltpu.sync_copy(aggS.at[sl], agg_out.at[cid].at[sl])
        return c

    lax.fori_loop(0, NROWS_SUB // ZB, dump, 0)


def _k4a(ef, rowi):
    return pl.kernel(
        _k4a_body,
        out_type=jax.ShapeDtypeStruct((NC, N_PAD, H), jnp.float32),
        mesh=_mesh(),
        compiler_params=pltpu.CompilerParams(needs_layout_passes=False),
        scratch_types=[
            pltpu.VMEM((BE,), jnp.int32),
            pltpu.VMEM((BE, H), jnp.float32),
            pltpu.VMEM((ZB, D), jnp.float32),
            pltpu.VMEM_SHARED((N_PAD, H), jnp.float32),
        ],
    )(ef, rowi)


def _k4b_body(mflat, rowi, coli, coordf, sums_out,
              rowv, colv, rowv8, mv, tbuf, zbuf, coordv, sbuf, ebuf, sumS):
    cid = lax.axis_index("c")
    sid = lax.axis_index("s")
    wid = sid * NC + cid
    base0 = wid * EW
    pltpu.sync_copy(coordf, coordv)

    zero16 = jnp.zeros((LANES,), jnp.float32)
    ones16 = jnp.ones((LANES,), jnp.float32)

    def zb_body(t, c):
        i = t // (D // LANES)
        k = lax.rem(t, D // LANES)
        zbuf[i, pl.ds(k * LANES, LANES)] = zero16
        return c

    lax.fori_loop(0, ZB * (D // LANES), zb_body, 0)

    def zt_body(t, c):
        i = t // (D // LANES)
        k = lax.rem(t, D // LANES)
        tbuf[i, pl.ds(k * LANES, LANES)] = zero16
        return c

    lax.fori_loop(0, BE * (D // LANES), zt_body, 0)

    def ze_body(t, c):
        i = t // (D // LANES)
        k = lax.rem(t, D // LANES)
        ebuf[i, pl.ds(k * LANES, LANES)] = zero16
        return c

    lax.fori_loop(0, 64 * (D // LANES), ze_body, 0)

    pltpu.sync_copy(zbuf.at[pl.ds(0, NSR // NS)], sumS.at[pl.ds(sid * (NSR // NS), NSR // NS)])
    plsc.subcore_barrier()

    def chunk(ci, carry):
        base = base0 + ci * BE
        pltpu.sync_copy(rowi.at[pl.ds(base, BE)], rowv)
        pltpu.sync_copy(coli.at[pl.ds(base, BE)], colv)
        pltpu.sync_copy(mflat.at[pl.ds(base, BE)], mv)

        def tb(g, c):
            sl = pl.ds(g * LANES, LANES)
            r = rowv[sl]
            r3 = r * 3
            c3 = colv[sl] * 3
            m16 = mv[sl]
            dx = _cgather(coordv, r3) - _cgather(coordv, c3)
            dy = _cgather(coordv, r3 + 1) - _cgather(coordv, c3 + 1)
            dz = _cgather(coordv, r3 + 2) - _cgather(coordv, c3 + 2)
            tx = jnp.clip(m16 * dx, -100.0, 100.0)
            ty = jnp.clip(m16 * dy, -100.0, 100.0)
            tz = jnp.clip(m16 * dz, -100.0, 100.0)
            rows = lax.iota(jnp.int32, LANES) + g * LANES
            rowv8[sl] = lax.shift_right_logical(r, 3)
            slot = (r & 7) * LANES
            plsc.store_scatter(tbuf, [rows, slot], tx)
            plsc.store_scatter(tbuf, [rows, slot + 1], ty)
            plsc.store_scatter(tbuf, [rows, slot + 2], tz)
            plsc.store_scatter(tbuf, [rows, slot + 3], ones16)
            return c

        lax.fori_loop(0, BE // LANES, tb, 0)
        pltpu.sync_copy(tbuf, sumS.at[rowv8], add=True)

        def tc(g, c):
            sl = pl.ds(g * LANES, LANES)
            r = rowv[sl]
            rows = lax.iota(jnp.int32, LANES) + g * LANES
            slot = (r & 7) * LANES
            plsc.store_scatter(tbuf, [rows, slot], zero16)
            plsc.store_scatter(tbuf, [rows, slot + 1], zero16)
            plsc.store_scatter(tbuf, [rows, slot + 2], zero16)
            plsc.store_scatter(tbuf, [rows, slot + 3], zero16)
            return c

        lax.fori_loop(0, BE // LANES, tc, 0)
        return carry

    lax.fori_loop(0, NCH, chunk, 0)
    plsc.subcore_barrier()

    def dumpb(j, c):
        nb = sid * NROWS_SUB + j * 64
        pltpu.sync_copy(sumS.at[pl.ds(sid * (NSR // NS) + j * 8, 8)], sbuf)

        def ex(g, cc):
            nloc = lax.iota(jnp.int32, LANES) + g * LANES
            srow = lax.shift_right_logical(nloc, 3)
            scol = (nloc & 7) * LANES
            for k in range(4):
                val = plsc.load_gather(sbuf, [srow, scol + k])
                plsc.store_scatter(ebuf, [nloc, jnp.full((LANES,), k, jnp.int32)], val)
            return cc

        lax.fori_loop(0, 64 // LANES, ex, 0)
        pltpu.sync_copy(ebuf, sums_out.at[cid].at[pl.ds(nb, 64)])
        return c

    lax.fori_loop(0, NROWS_SUB // 64, dumpb, 0)


def _k4b(mflat, rowi, coli, coordf):
    return pl.kernel(
        _k4b_body,
        out_type=jax.ShapeDtypeStruct((NC, N_PAD, D), jnp.float32),
        mesh=_mesh(),
        compiler_params=pltpu.CompilerParams(needs_layout_passes=False),
        scratch_types=[
            pltpu.VMEM((BE,), jnp.int32),
            pltpu.VMEM((BE,), jnp.int32),
            pltpu.VMEM((BE,), jnp.int32),
            pltpu.VMEM((BE,), jnp.float32),
            pltpu.VMEM((BE, D), jnp.float32),
            pltpu.VMEM((ZB, D), jnp.float32),
            pltpu.VMEM((CROWS, 128), jnp.float32),
            pltpu.VMEM((8, D), jnp.float32),
            pltpu.VMEM((64, D), jnp.float32),
            pltpu.VMEM_SHARED((NSR, D), jnp.float32),
        ],
    )(mflat, rowi, coli, coordf)


def _k5_body(h_ref, agg_ref, sums_ref, coord_ref, w1h_ref, w1a_ref, bn1_ref,
             wn2_ref, bn2_ref, hout_ref, cout_ref):
    agg = agg_ref[0] + agg_ref[1]
    hb = h_ref[...]
    t = jnp.maximum(
        _bdot(hb, w1h_ref[...]) + _bdot(agg, w1a_ref[...]) + bn1_ref[...], 0.0)
    hout_ref[...] = hb + _bdot(t, wn2_ref[...]) + bn2_ref[...]
    s = sums_ref[0] + sums_ref[1]
    cnt = jnp.maximum(s[:, 3:4], 1.0)
    cout_ref[...] = coord_ref[...] + s[:, 0:3] / cnt


def _k5(h, aggp, sumsp, coord, w1h, w1a, bn1, wn2, bn2):
    return pl.pallas_call(
        _k5_body,
        grid=(NNB5,),
        in_specs=[
            pl.BlockSpec((NB5, D), lambda i: (i, 0)),
            pl.BlockSpec((NC, NB5, H), lambda i: (0, i, 0)),
            pl.BlockSpec((NC, NB5, D), lambda i: (0, i, 0)),
            pl.BlockSpec((NB5, 3), lambda i: (i, 0)),
            pl.BlockSpec((D, H), lambda i: (0, 0)),
            pl.BlockSpec((H, H), lambda i: (0, 0)),
            pl.BlockSpec((1, H), lambda i: (0, 0)),
            pl.BlockSpec((H, D), lambda i: (0, 0)),
            pl.BlockSpec((1, D), lambda i: (0, 0)),
        ],
        out_specs=[
            pl.BlockSpec((NB5, D), lambda i: (i, 0)),
            pl.BlockSpec((NB5, 3), lambda i: (i, 0)),
        ],
        out_shape=[
            jax.ShapeDtypeStruct((N, D), jnp.float32),
            jax.ShapeDtypeStruct((N, 3), jnp.float32),
        ],
    )(h, aggp, sumsp, coord, w1h, w1a, bn1, wn2, bn2)


def kernel(h, edge_index, coord, edge_attr, We1, be1, We2, be2,
           Wn1, bn1, Wn2, bn2, Wc1, bc1, Wc2):
    row = edge_index[0]
    col = edge_index[1]
    wa = We1[0:D]
    wb = We1[D:2 * D]
    wr = We1[2 * D:2 * D + 1]
    wc = We1[2 * D + 1:]
    coordf = jnp.pad(coord.reshape(-1), (0, CROWS * 128 - 3 * N)).reshape(CROWS, 128)

    hwa, hwb = _k1(h, wa, wb)
    g, rad = _k2(hwa, hwb, row, col, coordf)
    rad2 = rad.reshape(NEB, 1, EB)
    ef, m2 = _k3(g, edge_attr, rad2, wc, wr, be1.reshape(1, H), We2,
                 be2.reshape(1, H), Wc1, bc1.reshape(1, H), Wc2)
    mflat = m2.reshape(-1)
    aggp = _k4a(ef, row)
    sumsp = _k4b(mflat, row, col, coordf)
    hout, cpart = _k5(h, aggp, sumsp, coord, Wn1[:D], Wn1[D:],
                      bn1.reshape(1, H), Wn2, bn2.reshape(1, D))
    return hout, cpart[:, :, None]


# speedup vs baseline: 5.9347x; 1.2076x over previous
import jax
import jax.numpy as jnp
from jax import lax
from jax.experimental import pallas as pl
from jax.experimental.pallas import tpu as pltpu
from jax.experimental.pallas import tpu_sc as plsc

N = 10000
E = 320000
D = 128
H = 128

NC = 2
NS = 16
NW = NC * NS
EW = E // NW
BE = 80
NCH = EW // BE
LANES = 16

EB = 2000
NEB = E // EB

NB = 1000
NNB = N // NB
NB5 = 1024
NNB5 = 10


CROWS = 240


def _bdot(a, b):
    return jnp.dot(a.astype(jnp.bfloat16), b.astype(jnp.bfloat16),
                   preferred_element_type=jnp.float32)


def _cgather(coordv, flat_idx):
    return plsc.load_gather(
        coordv, [lax.shift_right_logical(flat_idx, 7), flat_idx & 127])


def _mesh():
    return plsc.VectorSubcoreMesh(
        core_axis_name="c", subcore_axis_name="s", num_cores=NC, num_subcores=NS
    )


def _k1_body(h_ref, wa_ref, wb_ref, a_ref, b_ref):
    hb = h_ref[...]
    a_ref[...] = _bdot(hb, wa_ref[...])
    b_ref[...] = _bdot(hb, wb_ref[...])


def _k1(h, wa, wb):
    return pl.pallas_call(
        _k1_body,
        grid=(NNB,),
        in_specs=[
            pl.BlockSpec((NB, D), lambda i: (i, 0)),
            pl.BlockSpec((D, D), lambda i: (0, 0)),
            pl.BlockSpec((D, D), lambda i: (0, 0)),
        ],
        out_specs=[
            pl.BlockSpec((NB, D), lambda i: (i, 0)),
            pl.BlockSpec((NB, D), lambda i: (i, 0)),
        ],
        out_shape=[
            jax.ShapeDtypeStruct((N, D), jnp.float32),
            jax.ShapeDtypeStruct((N, D), jnp.float32),
        ],
    )(h, wa, wb)


def _k2_body(hwa, hwb, rowi, coli, coordf, g_out, rad_out,
             rowv0, rowv1, colv0, colv1, bufa0, bufa1, bufb0, bufb1,
             radv, coordv, sema0, sema1, semb0, semb1):
    cid = lax.axis_index("c")
    sid = lax.axis_index("s")
    wid = sid * NC + cid
    base0 = wid * EW
    pltpu.sync_copy(coordf, coordv)

    ROWV = [rowv0, rowv1]
    COLV = [colv0, colv1]
    BUFA = [bufa0, bufa1]
    BUFB = [bufb0, bufb1]
    SEMA = [sema0, sema1]
    SEMB = [semb0, semb1]

    def io(ci, b):
        base = base0 + ci * BE
        pltpu.sync_copy(rowi.at[pl.ds(base, BE)], ROWV[b])
        pltpu.sync_copy(coli.at[pl.ds(base, BE)], COLV[b])
        pltpu.async_copy(hwa.at[ROWV[b]], BUFA[b], SEMA[b])
        pltpu.async_copy(hwb.at[COLV[b]], BUFB[b], SEMB[b])

    def compute(ci, b):
        base = base0 + ci * BE
        pltpu.make_async_copy(hwa.at[ROWV[b]], BUFA[b], SEMA[b]).wait()
        pltpu.make_async_copy(hwb.at[COLV[b]], BUFB[b], SEMB[b]).wait()
        ba, bb = BUFA[b], BUFB[b]

        def add_body(i, c):
            for k in range(D // LANES):
                sl = pl.ds(k * LANES, LANES)
                ba[i, sl] = ba[i, sl] + bb[i, sl]
            return c

        lax.fori_loop(0, BE, add_body, 0)

        rv, cv = ROWV[b], COLV[b]

        def rad_body(g, c):
            sl = pl.ds(g * LANES, LANES)
            r3 = rv[sl] * 3
            c3 = cv[sl] * 3
            dx = _cgather(coordv, r3) - _cgather(coordv, c3)
            dy = _cgather(coordv, r3 + 1) - _cgather(coordv, c3 + 1)
            dz = _cgather(coordv, r3 + 2) - _cgather(coordv, c3 + 2)
            radv[sl] = dx * dx + dy * dy + dz * dz
            return c

        lax.fori_loop(0, BE // LANES, rad_body, 0)
        pltpu.sync_copy(ba, g_out.at[pl.ds(base, BE)])
        pltpu.sync_copy(radv, rad_out.at[pl.ds(base, BE)])

    io(0, 0)

    def pipe(j, carry):
        c0 = j * 2
        io(c0 + 1, 1)
        compute(c0, 0)
        io(c0 + 2, 0)
        compute(c0 + 1, 1)
        return carry

    lax.fori_loop(0, (NCH - 1) // 2, pipe, 0)
    compute(NCH - 1, 0)


def _k2(hwa, hwb, rowi, coli, coordf):
    return pl.kernel(
        _k2_body,
        out_type=[
            jax.ShapeDtypeStruct((E, D), jnp.float32),
            jax.ShapeDtypeStruct((E,), jnp.float32),
        ],
        mesh=_mesh(),
        compiler_params=pltpu.CompilerParams(needs_layout_passes=False),
        scratch_types=[
            pltpu.VMEM((BE,), jnp.int32),
            pltpu.VMEM((BE,), jnp.int32),
            pltpu.VMEM((BE,), jnp.int32),
            pltpu.VMEM((BE,), jnp.int32),
            pltpu.VMEM((BE, D), jnp.float32),
            pltpu.VMEM((BE, D), jnp.float32),
            pltpu.VMEM((BE, D), jnp.float32),
            pltpu.VMEM((BE, D), jnp.float32),
            pltpu.VMEM((BE,), jnp.float32),
            pltpu.VMEM((CROWS, 128), jnp.float32),
            pltpu.SemaphoreType.DMA,
            pltpu.SemaphoreType.DMA,
            pltpu.SemaphoreType.DMA,
            pltpu.SemaphoreType.DMA,
        ],
    )(hwa, hwb, rowi, coli, coordf)


def _k3_body(g_ref, ea_ref, rad_ref, wc_ref, wr_ref, be1_ref, w2_ref, be2_ref,
             wc1_ref, bc1_ref, wc2_ref, ef_ref, m_ref):
    rad_row = rad_ref[0]
    radp = lax.dot_general(rad_row, wr_ref[...], (((0,), (0,)), ((), ())),
                           preferred_element_type=jnp.float32)
    pre = (g_ref[...] + _bdot(ea_ref[...], wc_ref[...]) + radp + be1_ref[...])
    hdn = jnp.maximum(pre, 0.0)
    ef = jnp.maximum(_bdot(hdn, w2_ref[...]) + be2_ref[...], 0.0)
    ef_ref[...] = ef
    t = jnp.maximum(_bdot(ef, wc1_ref[...]) + bc1_ref[...], 0.0)
    m_ref[0] = lax.dot_general(wc2_ref[...], t, (((0,), (1,)), ((), ())),
                               preferred_element_type=jnp.float32)


def _k3(g, ea, rad2, wc, wr, be1, w2, be2, wc1, bc1, wc2):
    return pl.pallas_call(
        _k3_body,
        grid=(NEB,),
        in_specs=[
            pl.BlockSpec((EB, D), lambda i: (i, 0)),
            pl.BlockSpec((EB, H), lambda i: (i, 0)),
            pl.BlockSpec((1, 1, EB), lambda i: (i, 0, 0)),
            pl.BlockSpec((H, H), lambda i: (0, 0)),
            pl.BlockSpec((1, H), lambda i: (0, 0)),
            pl.BlockSpec((1, H), lambda i: (0, 0)),
            pl.BlockSpec((H, H), lambda i: (0, 0)),
            pl.BlockSpec((1, H), lambda i: (0, 0)),
            pl.BlockSpec((H, H), lambda i: (0, 0)),
            pl.BlockSpec((1, H), lambda i: (0, 0)),
            pl.BlockSpec((H, 1), lambda i: (0, 0)),
        ],
        out_specs=[
            pl.BlockSpec((EB, H), lambda i: (i, 0)),
            pl.BlockSpec((1, 1, EB), lambda i: (i, 0, 0)),
        ],
        out_shape=[
            jax.ShapeDtypeStruct((E, H), jnp.float32),
            jax.ShapeDtypeStruct((NEB, 1, EB), jnp.float32),
        ],
    )(g, ea, rad2, wc, wr, be1, w2, be2, wc1, bc1, wc2)


N_PAD = 10240
NROWS_SUB = N_PAD // NS
ZB = 128
NSR = N_PAD // 8


def _k4a_body(ef, rowi, agg_out, rowv, efv, zbuf, aggS):
    cid = lax.axis_index("c")
    sid = lax.axis_index("s")
    wid = sid * NC + cid
    base0 = wid * EW

    zero16 = jnp.zeros((LANES,), jnp.float32)

    def zb_body(t, c):
        i = t // (D // LANES)
        k = lax.rem(t, D // LANES)
        zbuf[i, pl.ds(k * LANES, LANES)] = zero16
        return c

    lax.fori_loop(0, ZB * (D // LANES), zb_body, 0)

    def zc_body(j, c):
        pltpu.sync_copy(zbuf, aggS.at[pl.ds(sid * NROWS_SUB + j * ZB, ZB)])
        return c

    lax.fori_loop(0, NROWS_SUB // ZB, zc_body, 0)
    plsc.subcore_barrier()

    def chunk(ci, carry):
        base = base0 + ci * BE
        pltpu.sync_copy(rowi.at[pl.ds(base, BE)], rowv)
        pltpu.sync_copy(ef.at[pl.ds(base, BE)], efv)
        pltpu.sync_copy(efv, aggS.at[rowv], add=True)
        return carry

    lax.fori_loop(0, NCH, chunk, 0)
    plsc.subcore_barrier()

    def dump(j, c):
        sl = pl.ds(sid * NROWS_SUB + j * ZB, ZB)
        pltpu.sync_copy(aggS.at[sl], agg_out.at[cid].at[sl])
        return c

    lax.fori_loop(0, NROWS_SUB // ZB, dump, 0)


def _k4a(ef, rowi):
    return pl.kernel(
        _k4a_body,
        out_type=jax.ShapeDtypeStruct((NC, N_PAD, H), jnp.float32),
        mesh=_mesh(),
        compiler_params=pltpu.CompilerParams(needs_layout_passes=False),
        scratch_types=[
            pltpu.VMEM((BE,), jnp.int32),
            pltpu.VMEM((BE, H), jnp.float32),
            pltpu.VMEM((ZB, D), jnp.float32),
            pltpu.VMEM_SHARED((N_PAD, H), jnp.float32),
        ],
    )(ef, rowi)


BE4 = 400


def _k4b_body(mflat, rowi, coli, coordf, sums_out,
              rw0, rw1, cw0, cw1, mw0, mw1, acc, pbuf, redbuf, ebuf, coordv,
              sstg, sem0, sem1):
    cid = lax.axis_index("c")
    sid = lax.axis_index("s")
    wid = sid * NC + cid
    base0 = wid * EW
    pltpu.sync_copy(coordf, coordv)

    RW = [rw0, rw1]
    CW = [cw0, cw1]
    MW = [mw0, mw1]
    SEM = [sem0, sem1]
    NCH4 = EW // BE4

    zero16 = jnp.zeros((LANES,), jnp.float32)
    ones16 = jnp.ones((LANES,), jnp.float32)

    def za_body(t, c):
        i = t // (D // LANES)
        k = lax.rem(t, D // LANES)
        acc[i, pl.ds(k * LANES, LANES)] = zero16
        return c

    lax.fori_loop(0, 320 * (D // LANES), za_body, 0)

    def ze_body(t, c):
        i = t // (D // LANES)
        k = lax.rem(t, D // LANES)
        ebuf[i, pl.ds(k * LANES, LANES)] = zero16
        return c

    lax.fori_loop(0, 32 * (D // LANES), ze_body, 0)

    def io(ci, b):
        base = base0 + ci * BE4
        pltpu.async_copy(rowi.at[pl.ds(base, BE4)], RW[b], SEM[b])
        pltpu.async_copy(coli.at[pl.ds(base, BE4)], CW[b], SEM[b])
        pltpu.async_copy(mflat.at[pl.ds(base, BE4)], MW[b], SEM[b])

    def compute(ci, b):
        base = base0 + ci * BE4
        pltpu.make_async_copy(rowi.at[pl.ds(base, BE4)], RW[b], SEM[b]).wait()
        pltpu.make_async_copy(coli.at[pl.ds(base, BE4)], CW[b], SEM[b]).wait()
        pltpu.make_async_copy(mflat.at[pl.ds(base, BE4)], MW[b], SEM[b]).wait()
        rw, cw, mw = RW[b], CW[b], MW[b]

        def grp(g, c):
            sl = pl.ds(g * LANES, LANES)
            r = rw[sl]
            r3 = r * 3
            c3 = cw[sl] * 3
            m16 = mw[sl]
            dx = _cgather(coordv, r3) - _cgather(coordv, c3)
            dy = _cgather(coordv, r3 + 1) - _cgather(coordv, c3 + 1)
            dz = _cgather(coordv, r3 + 2) - _cgather(coordv, c3 + 2)
            tx = jnp.clip(m16 * dx, -100.0, 100.0)
            ty = jnp.clip(m16 * dy, -100.0, 100.0)
            tz = jnp.clip(m16 * dz, -100.0, 100.0)
            arow = lax.shift_right_logical(r, 5)
            acol = (r & 31) * 4
            plsc.addupdate_scatter(acc, [arow, acol], tx)
            plsc.addupdate_scatter(acc, [arow, acol + 1], ty)
            plsc.addupdate_scatter(acc, [arow, acol + 2], tz)
            plsc.addupdate_scatter(acc, [arow, acol + 3], ones16)
            return c

        lax.fori_loop(0, BE4 // LANES, grp, 0)

    io(0, 0)

    def pipe(j, carry):
        c0 = j * 2
        io(c0 + 1, 1)
        compute(c0, 0)

        @pl.when(c0 + 2 < NCH4)
        def _():
            io(c0 + 2, 0)

        compute(c0 + 1, 1)
        return carry

    lax.fori_loop(0, NCH4 // 2, pipe, 0)
    compute(NCH4 - 1, 0)

    pltpu.sync_copy(acc, sstg.at[pl.ds(sid * 320, 320)])
    plsc.subcore_barrier()

    @pl.when(sid < 10)
    def _():
        def zr_body(t, c):
            i = t // (D // LANES)
            k = lax.rem(t, D // LANES)
            redbuf[i, pl.ds(k * LANES, LANES)] = zero16
            return c

        lax.fori_loop(0, 32 * (D // LANES), zr_body, 0)

        def red(p, c):
            pltpu.sync_copy(sstg.at[pl.ds(p * 320 + sid * 32, 32)], pbuf)

            def addp(t, cc):
                i = t // (D // LANES)
                k = lax.rem(t, D // LANES)
                slk = pl.ds(k * LANES, LANES)
                redbuf[i, slk] = redbuf[i, slk] + pbuf[i, slk]
                return cc

            lax.fori_loop(0, 32 * (D // LANES), addp, 0)
            return c

        lax.fori_loop(0, NS, red, 0)

        def batch(bi, c):
            def ex(g, cc):
                nloc = lax.iota(jnp.int32, LANES) + g * LANES
                srow = bi + lax.shift_right_logical(nloc, 5)
                scol = (nloc & 31) * 4
                for k in range(4):
                    val = plsc.load_gather(redbuf, [srow, scol + k])
                    plsc.store_scatter(ebuf, [nloc, jnp.full((LANES,), k, jnp.int32)], val)
                return cc

            lax.fori_loop(0, 32 // LANES, ex, 0)
            nb = sid * 1024 + bi * 32
            pltpu.sync_copy(ebuf, sums_out.at[cid].at[pl.ds(nb, 32)])
            return c

        lax.fori_loop(0, 32, batch, 0)


def _k4b(mflat, rowi, coli, coordf):
    return pl.kernel(
        _k4b_body,
        out_type=jax.ShapeDtypeStruct((NC, N_PAD, D), jnp.float32),
        mesh=_mesh(),
        compiler_params=pltpu.CompilerParams(needs_layout_passes=False),
        scratch_types=[
            pltpu.VMEM((BE4,), jnp.int32),
            pltpu.VMEM((BE4,), jnp.int32),
            pltpu.VMEM((BE4,), jnp.int32),
            pltpu.VMEM((BE4,), jnp.int32),
            pltpu.VMEM((BE4,), jnp.float32),
            pltpu.VMEM((BE4,), jnp.float32),
            pltpu.VMEM((320, D), jnp.float32),
            pltpu.VMEM((32, D), jnp.float32),
            pltpu.VMEM((32, D), jnp.float32),
            pltpu.VMEM((32, D), jnp.float32),
            pltpu.VMEM((CROWS, 128), jnp.float32),
            pltpu.VMEM_SHARED((NS * 320, D), jnp.float32),
            pltpu.SemaphoreType.DMA,
            pltpu.SemaphoreType.DMA,
        ],
    )(mflat, rowi, coli, coordf)


def _k5_body(h_ref, agg_ref, sums_ref, coord_ref, w1h_ref, w1a_ref, bn1_ref,
             wn2_ref, bn2_ref, hout_ref, cout_ref):
    agg = agg_ref[0] + agg_ref[1]
    hb = h_ref[...]
    t = jnp.maximum(
        _bdot(hb, w1h_ref[...]) + _bdot(agg, w1a_ref[...]) + bn1_ref[...], 0.0)
    hout_ref[...] = hb + _bdot(t, wn2_ref[...]) + bn2_ref[...]
    s = sums_ref[0] + sums_ref[1]
    cnt = jnp.maximum(s[:, 3:4], 1.0)
    cout_ref[...] = coord_ref[...] + s[:, 0:3] / cnt


def _k5(h, aggp, sumsp, coord, w1h, w1a, bn1, wn2, bn2):
    return pl.pallas_call(
        _k5_body,
        grid=(NNB5,),
        in_specs=[
            pl.BlockSpec((NB5, D), lambda i: (i, 0)),
            pl.BlockSpec((NC, NB5, H), lambda i: (0, i, 0)),
            pl.BlockSpec((NC, NB5, D), lambda i: (0, i, 0)),
            pl.BlockSpec((NB5, 3), lambda i: (i, 0)),
            pl.BlockSpec((D, H), lambda i: (0, 0)),
            pl.BlockSpec((H, H), lambda i: (0, 0)),
            pl.BlockSpec((1, H), lambda i: (0, 0)),
            pl.BlockSpec((H, D), lambda i: (0, 0)),
            pl.BlockSpec((1, D), lambda i: (0, 0)),
        ],
        out_specs=[
            pl.BlockSpec((NB5, D), lambda i: (i, 0)),
            pl.BlockSpec((NB5, 3), lambda i: (i, 0)),
        ],
        out_shape=[
            jax.ShapeDtypeStruct((N, D), jnp.float32),
            jax.ShapeDtypeStruct((N, 3), jnp.float32),
        ],
    )(h, aggp, sumsp, coord, w1h, w1a, bn1, wn2, bn2)


def kernel(h, edge_index, coord, edge_attr, We1, be1, We2, be2,
           Wn1, bn1, Wn2, bn2, Wc1, bc1, Wc2):
    row = edge_index[0]
    col = edge_index[1]
    wa = We1[0:D]
    wb = We1[D:2 * D]
    wr = We1[2 * D:2 * D + 1]
    wc = We1[2 * D + 1:]
    coordf = jnp.pad(coord.reshape(-1), (0, CROWS * 128 - 3 * N)).reshape(CROWS, 128)

    hwa, hwb = _k1(h, wa, wb)
    g, rad = _k2(hwa, hwb, row, col, coordf)
    rad2 = rad.reshape(NEB, 1, EB)
    ef, m2 = _k3(g, edge_attr, rad2, wc, wr, be1.reshape(1, H), We2,
                 be2.reshape(1, H), Wc1, bc1.reshape(1, H), Wc2)
    mflat = m2.reshape(-1)
    aggp = _k4a(ef, row)
    sumsp = _k4b(mflat, row, col, coordf)
    hout, cpart = _k5(h, aggp, sumsp, coord, Wn1[:D], Wn1[D:],
                      bn1.reshape(1, H), Wn2, bn2.reshape(1, D))
    return hout, cpart[:, :, None]


# speedup vs baseline: 6.9380x; 1.1691x over previous
import jax
import jax.numpy as jnp
from jax import lax
from jax.experimental import pallas as pl
from jax.experimental.pallas import tpu as pltpu
from jax.experimental.pallas import tpu_sc as plsc

N = 10000
E = 320000
D = 128
H = 128

NC = 2
NS = 16
NW = NC * NS
EW = E // NW
BE = 80
NCH = EW // BE
LANES = 16

EB = 2000
NEB = E // EB

NB = 1000
NNB = N // NB
NB5 = 1024
NNB5 = 10


CROWS = 240


def _bdot(a, b):
    return jnp.dot(a.astype(jnp.bfloat16), b.astype(jnp.bfloat16),
                   preferred_element_type=jnp.float32)


def _cgather(coordv, flat_idx):
    return plsc.load_gather(
        coordv, [lax.shift_right_logical(flat_idx, 7), flat_idx & 127])


def _mesh():
    return plsc.VectorSubcoreMesh(
        core_axis_name="c", subcore_axis_name="s", num_cores=NC, num_subcores=NS
    )


def _k1_body(h_ref, wa_ref, wb_ref, a_ref, b_ref):
    hb = h_ref[...]
    a_ref[...] = _bdot(hb, wa_ref[...])
    b_ref[...] = _bdot(hb, wb_ref[...])


def _k1(h, wa, wb):
    return pl.pallas_call(
        _k1_body,
        grid=(NNB,),
        in_specs=[
            pl.BlockSpec((NB, D), lambda i: (i, 0)),
            pl.BlockSpec((D, D), lambda i: (0, 0)),
            pl.BlockSpec((D, D), lambda i: (0, 0)),
        ],
        out_specs=[
            pl.BlockSpec((NB, D), lambda i: (i, 0)),
            pl.BlockSpec((NB, D), lambda i: (i, 0)),
        ],
        out_shape=[
            jax.ShapeDtypeStruct((N, D), jnp.float32),
            jax.ShapeDtypeStruct((N, D), jnp.float32),
        ],
    )(h, wa, wb)


def _k2_body(hwa, hwb, rowi, coli, coordf, g_out, rad_out,
             rowv0, rowv1, colv0, colv1, bufa0, bufa1, bufb0, bufb1,
             radv, coordv, sema0, sema1, semb0, semb1):
    cid = lax.axis_index("c")
    sid = lax.axis_index("s")
    wid = sid * NC + cid
    base0 = wid * EW
    pltpu.sync_copy(coordf, coordv)

    ROWV = [rowv0, rowv1]
    COLV = [colv0, colv1]
    BUFA = [bufa0, bufa1]
    BUFB = [bufb0, bufb1]
    SEMA = [sema0, sema1]
    SEMB = [semb0, semb1]

    def io(ci, b):
        base = base0 + ci * BE
        pltpu.sync_copy(rowi.at[pl.ds(base, BE)], ROWV[b])
        pltpu.sync_copy(coli.at[pl.ds(base, BE)], COLV[b])
        pltpu.async_copy(hwa.at[ROWV[b]], BUFA[b], SEMA[b])
        pltpu.async_copy(hwb.at[COLV[b]], BUFB[b], SEMB[b])

    def compute(ci, b):
        base = base0 + ci * BE
        pltpu.make_async_copy(hwa.at[ROWV[b]], BUFA[b], SEMA[b]).wait()
        pltpu.make_async_copy(hwb.at[COLV[b]], BUFB[b], SEMB[b]).wait()
        ba, bb = BUFA[b], BUFB[b]

        def add_body(i, c):
            for k in range(D // LANES):
                sl = pl.ds(k * LANES, LANES)
                ba[i, sl] = ba[i, sl] + bb[i, sl]
            return c

        lax.fori_loop(0, BE, add_body, 0)

        rv, cv = ROWV[b], COLV[b]

        def rad_body(g, c):
            sl = pl.ds(g * LANES, LANES)
            r3 = rv[sl] * 3
            c3 = cv[sl] * 3
            dx = _cgather(coordv, r3) - _cgather(coordv, c3)
            dy = _cgather(coordv, r3 + 1) - _cgather(coordv, c3 + 1)
            dz = _cgather(coordv, r3 + 2) - _cgather(coordv, c3 + 2)
            radv[sl] = dx * dx + dy * dy + dz * dz
            return c

        lax.fori_loop(0, BE // LANES, rad_body, 0)
        pltpu.sync_copy(ba, g_out.at[pl.ds(base, BE)])
        pltpu.sync_copy(radv, rad_out.at[pl.ds(base, BE)])

    io(0, 0)

    def pipe(j, carry):
        c0 = j * 2
        io(c0 + 1, 1)
        compute(c0, 0)
        io(c0 + 2, 0)
        compute(c0 + 1, 1)
        return carry

    lax.fori_loop(0, (NCH - 1) // 2, pipe, 0)
    compute(NCH - 1, 0)


def _k2(hwa, hwb, rowi, coli, coordf):
    return pl.kernel(
        _k2_body,
        out_type=[
            jax.ShapeDtypeStruct((E, D), jnp.float32),
            jax.ShapeDtypeStruct((E,), jnp.float32),
        ],
        mesh=_mesh(),
        compiler_params=pltpu.CompilerParams(needs_layout_passes=False),
        scratch_types=[
            pltpu.VMEM((BE,), jnp.int32),
            pltpu.VMEM((BE,), jnp.int32),
            pltpu.VMEM((BE,), jnp.int32),
            pltpu.VMEM((BE,), jnp.int32),
            pltpu.VMEM((BE, D), jnp.float32),
            pltpu.VMEM((BE, D), jnp.float32),
            pltpu.VMEM((BE, D), jnp.float32),
            pltpu.VMEM((BE, D), jnp.float32),
            pltpu.VMEM((BE,), jnp.float32),
            pltpu.VMEM((CROWS, 128), jnp.float32),
            pltpu.SemaphoreType.DMA,
            pltpu.SemaphoreType.DMA,
            pltpu.SemaphoreType.DMA,
            pltpu.SemaphoreType.DMA,
        ],
    )(hwa, hwb, rowi, coli, coordf)


def _k3_body(g_ref, ea_ref, rad_ref, wc_ref, wr_ref, be1_ref, w2_ref, be2_ref,
             wc1_ref, bc1_ref, wc2_ref, ef_ref, m_ref):
    rad_row = rad_ref[0]
    radp = lax.dot_general(rad_row, wr_ref[...], (((0,), (0,)), ((), ())),
                           preferred_element_type=jnp.float32)
    pre = (g_ref[...] + _bdot(ea_ref[...], wc_ref[...]) + radp + be1_ref[...])
    hdn = jnp.maximum(pre, 0.0)
    ef = jnp.maximum(_bdot(hdn, w2_ref[...]) + be2_ref[...], 0.0)
    ef_ref[...] = ef
    t = jnp.maximum(_bdot(ef, wc1_ref[...]) + bc1_ref[...], 0.0)
    m_ref[0] = lax.dot_general(wc2_ref[...], t, (((0,), (1,)), ((), ())),
                               preferred_element_type=jnp.float32)


def _k3(g, ea, rad2, wc, wr, be1, w2, be2, wc1, bc1, wc2):
    return pl.pallas_call(
        _k3_body,
        grid=(NEB,),
        in_specs=[
            pl.BlockSpec((EB, D), lambda i: (i, 0)),
            pl.BlockSpec((EB, H), lambda i: (i, 0)),
            pl.BlockSpec((1, 1, EB), lambda i: (i, 0, 0)),
            pl.BlockSpec((H, H), lambda i: (0, 0)),
            pl.BlockSpec((1, H), lambda i: (0, 0)),
            pl.BlockSpec((1, H), lambda i: (0, 0)),
            pl.BlockSpec((H, H), lambda i: (0, 0)),
            pl.BlockSpec((1, H), lambda i: (0, 0)),
            pl.BlockSpec((H, H), lambda i: (0, 0)),
            pl.BlockSpec((1, H), lambda i: (0, 0)),
            pl.BlockSpec((H, 1), lambda i: (0, 0)),
        ],
        out_specs=[
            pl.BlockSpec((EB, H), lambda i: (i, 0)),
            pl.BlockSpec((1, 1, EB), lambda i: (i, 0, 0)),
        ],
        out_shape=[
            jax.ShapeDtypeStruct((E, H), jnp.float32),
            jax.ShapeDtypeStruct((NEB, 1, EB), jnp.float32),
        ],
    )(g, ea, rad2, wc, wr, be1, w2, be2, wc1, bc1, wc2)


N_PAD = 10240
NROWS_SUB = N_PAD // NS
ZB = 128
NSR = N_PAD // 8


def _k4a_body(ef, rowi, agg_out, rw0, rw1, ef0, ef1, zbuf, aggS, sem0, sem1):
    cid = lax.axis_index("c")
    sid = lax.axis_index("s")
    wid = sid * NC + cid
    base0 = wid * EW

    RW = [rw0, rw1]
    EFV = [ef0, ef1]
    SEM = [sem0, sem1]

    zero16 = jnp.zeros((LANES,), jnp.float32)

    def zb_body(t, c):
        i = t // (D // LANES)
        k = lax.rem(t, D // LANES)
        zbuf[i, pl.ds(k * LANES, LANES)] = zero16
        return c

    lax.fori_loop(0, ZB * (D // LANES), zb_body, 0)

    def zc_body(j, c):
        pltpu.sync_copy(zbuf, aggS.at[pl.ds(sid * NROWS_SUB + j * ZB, ZB)])
        return c

    lax.fori_loop(0, NROWS_SUB // ZB, zc_body, 0)
    plsc.subcore_barrier()

    def io(ci, b):
        base = base0 + ci * BE
        pltpu.async_copy(rowi.at[pl.ds(base, BE)], RW[b], SEM[b])
        pltpu.async_copy(ef.at[pl.ds(base, BE)], EFV[b], SEM[b])

    def scat(ci, b):
        base = base0 + ci * BE
        pltpu.make_async_copy(rowi.at[pl.ds(base, BE)], RW[b], SEM[b]).wait()
        pltpu.make_async_copy(ef.at[pl.ds(base, BE)], EFV[b], SEM[b]).wait()
        pltpu.sync_copy(EFV[b], aggS.at[RW[b]], add=True)

    io(0, 0)

    def pipe(j, carry):
        c0 = j * 2
        io(c0 + 1, 1)
        scat(c0, 0)
        io(c0 + 2, 0)
        scat(c0 + 1, 1)
        return carry

    lax.fori_loop(0, (NCH - 1) // 2, pipe, 0)
    scat(NCH - 1, 0)
    plsc.subcore_barrier()

    def dump(j, c):
        sl = pl.ds(sid * NROWS_SUB + j * ZB, ZB)
        pltpu.sync_copy(aggS.at[sl], agg_out.at[cid].at[sl])
        return c

    lax.fori_loop(0, NROWS_SUB // ZB, dump, 0)


def _k4a(ef, rowi):
    return pl.kernel(
        _k4a_body,
        out_type=jax.ShapeDtypeStruct((NC, N_PAD, H), jnp.float32),
        mesh=_mesh(),
        compiler_params=pltpu.CompilerParams(needs_layout_passes=False),
        scratch_types=[
            pltpu.VMEM((BE,), jnp.int32),
            pltpu.VMEM((BE,), jnp.int32),
            pltpu.VMEM((BE, H), jnp.float32),
            pltpu.VMEM((BE, H), jnp.float32),
            pltpu.VMEM((ZB, D), jnp.float32),
            pltpu.VMEM_SHARED((N_PAD, H), jnp.float32),
            pltpu.SemaphoreType.DMA,
            pltpu.SemaphoreType.DMA,
        ],
    )(ef, rowi)


BE4 = 400


def _k4b_body(mflat, rowi, coli, coordf, sums_out,
              rw0, rw1, cw0, cw1, mw0, mw1, acc, pbuf, redbuf, ebuf, coordv,
              sstg, sem0, sem1):
    cid = lax.axis_index("c")
    sid = lax.axis_index("s")
    wid = sid * NC + cid
    base0 = wid * EW
    pltpu.sync_copy(coordf, coordv)

    RW = [rw0, rw1]
    CW = [cw0, cw1]
    MW = [mw0, mw1]
    SEM = [sem0, sem1]
    NCH4 = EW // BE4

    zero16 = jnp.zeros((LANES,), jnp.float32)
    ones16 = jnp.ones((LANES,), jnp.float32)

    def za_body(t, c):
        i = t // (D // LANES)
        k = lax.rem(t, D // LANES)
        acc[i, pl.ds(k * LANES, LANES)] = zero16
        return c

    lax.fori_loop(0, 320 * (D // LANES), za_body, 0)

    def ze_body(t, c):
        i = t // (D // LANES)
        k = lax.rem(t, D // LANES)
        ebuf[i, pl.ds(k * LANES, LANES)] = zero16
        return c

    lax.fori_loop(0, 32 * (D // LANES), ze_body, 0)

    def io(ci, b):
        base = base0 + ci * BE4
        pltpu.async_copy(rowi.at[pl.ds(base, BE4)], RW[b], SEM[b])
        pltpu.async_copy(coli.at[pl.ds(base, BE4)], CW[b], SEM[b])
        pltpu.async_copy(mflat.at[pl.ds(base, BE4)], MW[b], SEM[b])

    def compute(ci, b):
        base = base0 + ci * BE4
        pltpu.make_async_copy(rowi.at[pl.ds(base, BE4)], RW[b], SEM[b]).wait()
        pltpu.make_async_copy(coli.at[pl.ds(base, BE4)], CW[b], SEM[b]).wait()
        pltpu.make_async_copy(mflat.at[pl.ds(base, BE4)], MW[b], SEM[b]).wait()
        rw, cw, mw = RW[b], CW[b], MW[b]

        def grp(g, c):
            sl = pl.ds(g * LANES, LANES)
            r = rw[sl]
            r3 = r * 3
            c3 = cw[sl] * 3
            m16 = mw[sl]
            dx = _cgather(coordv, r3) - _cgather(coordv, c3)
            dy = _cgather(coordv, r3 + 1) - _cgather(coordv, c3 + 1)
            dz = _cgather(coordv, r3 + 2) - _cgather(coordv, c3 + 2)
            tx = jnp.clip(m16 * dx, -100.0, 100.0)
            ty = jnp.clip(m16 * dy, -100.0, 100.0)
            tz = jnp.clip(m16 * dz, -100.0, 100.0)
            arow = lax.shift_right_logical(r, 5)
            acol = (r & 31) * 4
            plsc.addupdate_scatter(acc, [arow, acol], tx)
            plsc.addupdate_scatter(acc, [arow, acol + 1], ty)
            plsc.addupdate_scatter(acc, [arow, acol + 2], tz)
            plsc.addupdate_scatter(acc, [arow, acol + 3], ones16)
            return c

        lax.fori_loop(0, BE4 // LANES, grp, 0)

    io(0, 0)

    def pipe(j, carry):
        c0 = j * 2
        io(c0 + 1, 1)
        compute(c0, 0)

        @pl.when(c0 + 2 < NCH4)
        def _():
            io(c0 + 2, 0)

        compute(c0 + 1, 1)
        return carry

    lax.fori_loop(0, NCH4 // 2, pipe, 0)
    compute(NCH4 - 1, 0)

    pltpu.sync_copy(acc, sstg.at[pl.ds(sid * 320, 320)])
    plsc.subcore_barrier()

    @pl.when(sid < 10)
    def _():
        def zr_body(t, c):
            i = t // (D // LANES)
            k = lax.rem(t, D // LANES)
            redbuf[i, pl.ds(k * LANES, LANES)] = zero16
            return c

        lax.fori_loop(0, 32 * (D // LANES), zr_body, 0)

        def red(p, c):
            pltpu.sync_copy(sstg.at[pl.ds(p * 320 + sid * 32, 32)], pbuf)

            def addp(t, cc):
                i = t // (D // LANES)
                k = lax.rem(t, D // LANES)
                slk = pl.ds(k * LANES, LANES)
                redbuf[i, slk] = redbuf[i, slk] + pbuf[i, slk]
                return cc

            lax.fori_loop(0, 32 * (D // LANES), addp, 0)
            return c

        lax.fori_loop(0, NS, red, 0)

        def batch(bi, c):
            def ex(g, cc):
                nloc = lax.iota(jnp.int32, LANES) + g * LANES
                srow = bi + lax.shift_right_logical(nloc, 5)
                scol = (nloc & 31) * 4
                for k in range(4):
                    val = plsc.load_gather(redbuf, [srow, scol + k])
                    plsc.store_scatter(ebuf, [nloc, jnp.full((LANES,), k, jnp.int32)], val)
                return cc

            lax.fori_loop(0, 32 // LANES, ex, 0)
            nb = sid * 1024 + bi * 32
            pltpu.sync_copy(ebuf, sums_out.at[cid].at[pl.ds(nb, 32)])
            return c

        lax.fori_loop(0, 32, batch, 0)


def _k4b(mflat, rowi, coli, coordf):
    return pl.kernel(
        _k4b_body,
        out_type=jax.ShapeDtypeStruct((NC, N_PAD, D), jnp.float32),
        mesh=_mesh(),
        compiler_params=pltpu.CompilerParams(needs_layout_passes=False),
        scratch_types=[
            pltpu.VMEM((BE4,), jnp.int32),
            pltpu.VMEM((BE4,), jnp.int32),
            pltpu.VMEM((BE4,), jnp.int32),
            pltpu.VMEM((BE4,), jnp.int32),
            pltpu.VMEM((BE4,), jnp.float32),
            pltpu.VMEM((BE4,), jnp.float32),
            pltpu.VMEM((320, D), jnp.float32),
            pltpu.VMEM((32, D), jnp.float32),
            pltpu.VMEM((32, D), jnp.float32),
            pltpu.VMEM((32, D), jnp.float32),
            pltpu.VMEM((CROWS, 128), jnp.float32),
            pltpu.VMEM_SHARED((NS * 320, D), jnp.float32),
            pltpu.SemaphoreType.DMA,
            pltpu.SemaphoreType.DMA,
        ],
    )(mflat, rowi, coli, coordf)


def _k5_body(h_ref, agg_ref, sums_ref, coord_ref, w1h_ref, w1a_ref, bn1_ref,
             wn2_ref, bn2_ref, hout_ref, cout_ref):
    agg = agg_ref[0] + agg_ref[1]
    hb = h_ref[...]
    t = jnp.maximum(
        _bdot(hb, w1h_ref[...]) + _bdot(agg, w1a_ref[...]) + bn1_ref[...], 0.0)
    hout_ref[...] = hb + _bdot(t, wn2_ref[...]) + bn2_ref[...]
    s = sums_ref[0] + sums_ref[1]
    cnt = jnp.maximum(s[:, 3:4], 1.0)
    cout_ref[...] = coord_ref[...] + s[:, 0:3] / cnt


def _k5(h, aggp, sumsp, coord, w1h, w1a, bn1, wn2, bn2):
    return pl.pallas_call(
        _k5_body,
        grid=(NNB5,),
        in_specs=[
            pl.BlockSpec((NB5, D), lambda i: (i, 0)),
            pl.BlockSpec((NC, NB5, H), lambda i: (0, i, 0)),
            pl.BlockSpec((NC, NB5, D), lambda i: (0, i, 0)),
            pl.BlockSpec((NB5, 3), lambda i: (i, 0)),
            pl.BlockSpec((D, H), lambda i: (0, 0)),
            pl.BlockSpec((H, H), lambda i: (0, 0)),
            pl.BlockSpec((1, H), lambda i: (0, 0)),
            pl.BlockSpec((H, D), lambda i: (0, 0)),
            pl.BlockSpec((1, D), lambda i: (0, 0)),
        ],
        out_specs=[
            pl.BlockSpec((NB5, D), lambda i: (i, 0)),
            pl.BlockSpec((NB5, 3), lambda i: (i, 0)),
        ],
        out_shape=[
            jax.ShapeDtypeStruct((N, D), jnp.float32),
            jax.ShapeDtypeStruct((N, 3), jnp.float32),
        ],
    )(h, aggp, sumsp, coord, w1h, w1a, bn1, wn2, bn2)


def kernel(h, edge_index, coord, edge_attr, We1, be1, We2, be2,
           Wn1, bn1, Wn2, bn2, Wc1, bc1, Wc2):
    row = edge_index[0]
    col = edge_index[1]
    wa = We1[0:D]
    wb = We1[D:2 * D]
    wr = We1[2 * D:2 * D + 1]
    wc = We1[2 * D + 1:]
    coordf = jnp.pad(coord.reshape(-1), (0, CROWS * 128 - 3 * N)).reshape(CROWS, 128)

    hwa, hwb = _k1(h, wa, wb)
    g, rad = _k2(hwa, hwb, row, col, coordf)
    rad2 = rad.reshape(NEB, 1, EB)
    ef, m2 = _k3(g, edge_attr, rad2, wc, wr, be1.reshape(1, H), We2,
                 be2.reshape(1, H), Wc1, bc1.reshape(1, H), Wc2)
    mflat = m2.reshape(-1)
    aggp = _k4a(ef, row)
    sumsp = _k4b(mflat, row, col, coordf)
    hout, cpart = _k5(h, aggp, sumsp, coord, Wn1[:D], Wn1[D:],
                      bn1.reshape(1, H), Wn2, bn2.reshape(1, D))
    return hout, cpart[:, :, None]


# speedup vs baseline: 7.6124x; 1.0972x over previous
import jax
import jax.numpy as jnp
from jax import lax
from jax.experimental import pallas as pl
from jax.experimental.pallas import tpu as pltpu
from jax.experimental.pallas import tpu_sc as plsc

N = 10000
E = 320000
D = 128
H = 128

NC = 2
NS = 16
NW = NC * NS
EW = E // NW
BE = 80
NCH = EW // BE
LANES = 16

EB = 4000
NEB = E // EB

NB = 1000
NNB = N // NB
NB5 = 1024
NNB5 = 10


CROWS = 240


def _bdot(a, b):
    return jnp.dot(a.astype(jnp.bfloat16), b.astype(jnp.bfloat16),
                   preferred_element_type=jnp.float32)


def _cgather(coordv, flat_idx):
    return plsc.load_gather(
        coordv, [lax.shift_right_logical(flat_idx, 7), flat_idx & 127])


def _mesh():
    return plsc.VectorSubcoreMesh(
        core_axis_name="c", subcore_axis_name="s", num_cores=NC, num_subcores=NS
    )


def _k1_body(h_ref, wa_ref, wb_ref, a_ref, b_ref):
    hb = h_ref[...]
    a_ref[...] = _bdot(hb, wa_ref[...])
    b_ref[...] = _bdot(hb, wb_ref[...])


def _k1(h, wa, wb):
    return pl.pallas_call(
        _k1_body,
        grid=(NNB,),
        in_specs=[
            pl.BlockSpec((NB, D), lambda i: (i, 0)),
            pl.BlockSpec((D, D), lambda i: (0, 0)),
            pl.BlockSpec((D, D), lambda i: (0, 0)),
        ],
        out_specs=[
            pl.BlockSpec((NB, D), lambda i: (i, 0)),
            pl.BlockSpec((NB, D), lambda i: (i, 0)),
        ],
        out_shape=[
            jax.ShapeDtypeStruct((N, D), jnp.float32),
            jax.ShapeDtypeStruct((N, D), jnp.float32),
        ],
    )(h, wa, wb)


def _k2_body(hwa, hwb, rowi, coli, coordf, g_out, rad_out,
             rowv0, rowv1, colv0, colv1, bufa0, bufa1, bufb0, bufb1,
             radv, coordv, sema0, sema1, semb0, semb1):
    cid = lax.axis_index("c")
    sid = lax.axis_index("s")
    wid = sid * NC + cid
    base0 = wid * EW
    pltpu.sync_copy(coordf, coordv)

    ROWV = [rowv0, rowv1]
    COLV = [colv0, colv1]
    BUFA = [bufa0, bufa1]
    BUFB = [bufb0, bufb1]
    SEMA = [sema0, sema1]
    SEMB = [semb0, semb1]

    def io(ci, b):
        base = base0 + ci * BE
        pltpu.sync_copy(rowi.at[pl.ds(base, BE)], ROWV[b])
        pltpu.sync_copy(coli.at[pl.ds(base, BE)], COLV[b])
        pltpu.async_copy(hwa.at[ROWV[b]], BUFA[b], SEMA[b])
        pltpu.async_copy(hwb.at[COLV[b]], BUFB[b], SEMB[b])

    def compute(ci, b):
        base = base0 + ci * BE
        pltpu.make_async_copy(hwa.at[ROWV[b]], BUFA[b], SEMA[b]).wait()
        pltpu.make_async_copy(hwb.at[COLV[b]], BUFB[b], SEMB[b]).wait()
        ba, bb = BUFA[b], BUFB[b]

        def add_body(i, c):
            for k in range(D // LANES):
                sl = pl.ds(k * LANES, LANES)
                ba[i, sl] = ba[i, sl] + bb[i, sl]
            return c

        lax.fori_loop(0, BE, add_body, 0)

        rv, cv = ROWV[b], COLV[b]

        def rad_body(g, c):
            sl = pl.ds(g * LANES, LANES)
            r3 = rv[sl] * 3
            c3 = cv[sl] * 3
            dx = _cgather(coordv, r3) - _cgather(coordv, c3)
            dy = _cgather(coordv, r3 + 1) - _cgather(coordv, c3 + 1)
            dz = _cgather(coordv, r3 + 2) - _cgather(coordv, c3 + 2)
            radv[sl] = dx * dx + dy * dy + dz * dz
            return c

        lax.fori_loop(0, BE // LANES, rad_body, 0)
        pltpu.sync_copy(ba, g_out.at[pl.ds(base, BE)])
        pltpu.sync_copy(radv, rad_out.at[pl.ds(base, BE)])

    io(0, 0)

    def pipe(j, carry):
        c0 = j * 2
        io(c0 + 1, 1)
        compute(c0, 0)
        io(c0 + 2, 0)
        compute(c0 + 1, 1)
        return carry

    lax.fori_loop(0, (NCH - 1) // 2, pipe, 0)
    compute(NCH - 1, 0)


def _k2(hwa, hwb, rowi, coli, coordf):
    return pl.kernel(
        _k2_body,
        out_type=[
            jax.ShapeDtypeStruct((E, D), jnp.float32),
            jax.ShapeDtypeStruct((E,), jnp.float32),
        ],
        mesh=_mesh(),
        compiler_params=pltpu.CompilerParams(needs_layout_passes=False),
        scratch_types=[
            pltpu.VMEM((BE,), jnp.int32),
            pltpu.VMEM((BE,), jnp.int32),
            pltpu.VMEM((BE,), jnp.int32),
            pltpu.VMEM((BE,), jnp.int32),
            pltpu.VMEM((BE, D), jnp.float32),
            pltpu.VMEM((BE, D), jnp.float32),
            pltpu.VMEM((BE, D), jnp.float32),
            pltpu.VMEM((BE, D), jnp.float32),
            pltpu.VMEM((BE,), jnp.float32),
            pltpu.VMEM((CROWS, 128), jnp.float32),
            pltpu.SemaphoreType.DMA,
            pltpu.SemaphoreType.DMA,
            pltpu.SemaphoreType.DMA,
            pltpu.SemaphoreType.DMA,
        ],
    )(hwa, hwb, rowi, coli, coordf)


def _k3_body(g_ref, ea_ref, rad_ref, wc_ref, wr_ref, be1_ref, w2_ref, be2_ref,
             wc1_ref, bc1_ref, wc2_ref, ef_ref, m_ref):
    rad_row = rad_ref[0]
    radp = lax.dot_general(rad_row, wr_ref[...], (((0,), (0,)), ((), ())),
                           preferred_element_type=jnp.float32)
    pre = (g_ref[...] + _bdot(ea_ref[...], wc_ref[...]) + radp + be1_ref[...])
    hdn = jnp.maximum(pre, 0.0)
    ef = jnp.maximum(_bdot(hdn, w2_ref[...]) + be2_ref[...], 0.0)
    ef_ref[...] = ef
    t = jnp.maximum(_bdot(ef, wc1_ref[...]) + bc1_ref[...], 0.0)
    m_ref[0] = lax.dot_general(wc2_ref[...], t, (((0,), (1,)), ((), ())),
                               preferred_element_type=jnp.float32)


def _k3(g, ea, rad2, wc, wr, be1, w2, be2, wc1, bc1, wc2):
    return pl.pallas_call(
        _k3_body,
        grid=(NEB,),
        in_specs=[
            pl.BlockSpec((EB, D), lambda i: (i, 0)),
            pl.BlockSpec((EB, H), lambda i: (i, 0)),
            pl.BlockSpec((1, 1, EB), lambda i: (i, 0, 0)),
            pl.BlockSpec((H, H), lambda i: (0, 0)),
            pl.BlockSpec((1, H), lambda i: (0, 0)),
            pl.BlockSpec((1, H), lambda i: (0, 0)),
            pl.BlockSpec((H, H), lambda i: (0, 0)),
            pl.BlockSpec((1, H), lambda i: (0, 0)),
            pl.BlockSpec((H, H), lambda i: (0, 0)),
            pl.BlockSpec((1, H), lambda i: (0, 0)),
            pl.BlockSpec((H, 1), lambda i: (0, 0)),
        ],
        out_specs=[
            pl.BlockSpec((EB, H), lambda i: (i, 0)),
            pl.BlockSpec((1, 1, EB), lambda i: (i, 0, 0)),
        ],
        out_shape=[
            jax.ShapeDtypeStruct((E, H), jnp.float32),
            jax.ShapeDtypeStruct((NEB, 1, EB), jnp.float32),
        ],
    )(g, ea, rad2, wc, wr, be1, w2, be2, wc1, bc1, wc2)


N_PAD = 10240
NROWS_SUB = N_PAD // NS
ZB = 128
NSR = N_PAD // 8


def _k4a_body(ef, rowi, agg_out, rw0, rw1, ef0, ef1, zbuf, aggS, sem0, sem1):
    cid = lax.axis_index("c")
    sid = lax.axis_index("s")
    wid = sid * NC + cid
    base0 = wid * EW

    RW = [rw0, rw1]
    EFV = [ef0, ef1]
    SEM = [sem0, sem1]

    zero16 = jnp.zeros((LANES,), jnp.float32)

    def zb_body(t, c):
        i = t // (D // LANES)
        k = lax.rem(t, D // LANES)
        zbuf[i, pl.ds(k * LANES, LANES)] = zero16
        return c

    lax.fori_loop(0, ZB * (D // LANES), zb_body, 0)

    def zc_body(j, c):
        pltpu.sync_copy(zbuf, aggS.at[pl.ds(sid * NROWS_SUB + j * ZB, ZB)])
        return c

    lax.fori_loop(0, NROWS_SUB // ZB, zc_body, 0)
    plsc.subcore_barrier()

    def io(ci, b):
        base = base0 + ci * BE
        pltpu.async_copy(rowi.at[pl.ds(base, BE)], RW[b], SEM[b])
        pltpu.async_copy(ef.at[pl.ds(base, BE)], EFV[b], SEM[b])

    def scat(ci, b):
        base = base0 + ci * BE
        pltpu.make_async_copy(rowi.at[pl.ds(base, BE)], RW[b], SEM[b]).wait()
        pltpu.make_async_copy(ef.at[pl.ds(base, BE)], EFV[b], SEM[b]).wait()
        pltpu.sync_copy(EFV[b], aggS.at[RW[b]], add=True)

    io(0, 0)

    def pipe(j, carry):
        c0 = j * 2
        io(c0 + 1, 1)
        scat(c0, 0)
        io(c0 + 2, 0)
        scat(c0 + 1, 1)
        return carry

    lax.fori_loop(0, (NCH - 1) // 2, pipe, 0)
    scat(NCH - 1, 0)
    plsc.subcore_barrier()

    def dump(j, c):
        sl = pl.ds(sid * NROWS_SUB + j * ZB, ZB)
        pltpu.sync_copy(aggS.at[sl], agg_out.at[cid].at[sl])
        return c

    lax.fori_loop(0, NROWS_SUB // ZB, dump, 0)


def _k4a(ef, rowi):
    return pl.kernel(
        _k4a_body,
        out_type=jax.ShapeDtypeStruct((NC, N_PAD, H), jnp.float32),
        mesh=_mesh(),
        compiler_params=pltpu.CompilerParams(needs_layout_passes=False),
        scratch_types=[
            pltpu.VMEM((BE,), jnp.int32),
            pltpu.VMEM((BE,), jnp.int32),
            pltpu.VMEM((BE, H), jnp.float32),
            pltpu.VMEM((BE, H), jnp.float32),
            pltpu.VMEM((ZB, D), jnp.float32),
            pltpu.VMEM_SHARED((N_PAD, H), jnp.float32),
            pltpu.SemaphoreType.DMA,
            pltpu.SemaphoreType.DMA,
        ],
    )(ef, rowi)


BE4 = 400


def _k4b_body(mflat, rowi, coli, coordf, sums_out,
              rw0, rw1, cw0, cw1, mw0, mw1, acc, pbuf, redbuf, ebuf, coordv,
              sstg, sem0, sem1):
    cid = lax.axis_index("c")
    sid = lax.axis_index("s")
    wid = sid * NC + cid
    base0 = wid * EW
    pltpu.sync_copy(coordf, coordv)

    RW = [rw0, rw1]
    CW = [cw0, cw1]
    MW = [mw0, mw1]
    SEM = [sem0, sem1]
    NCH4 = EW // BE4

    zero16 = jnp.zeros((LANES,), jnp.float32)
    ones16 = jnp.ones((LANES,), jnp.float32)

    def za_body(t, c):
        i = t // (D // LANES)
        k = lax.rem(t, D // LANES)
        acc[i, pl.ds(k * LANES, LANES)] = zero16
        return c

    lax.fori_loop(0, 320 * (D // LANES), za_body, 0)

    def ze_body(t, c):
        i = t // (D // LANES)
        k = lax.rem(t, D // LANES)
        ebuf[i, pl.ds(k * LANES, LANES)] = zero16
        return c

    lax.fori_loop(0, 32 * (D // LANES), ze_body, 0)

    def io(ci, b):
        base = base0 + ci * BE4
        pltpu.async_copy(rowi.at[pl.ds(base, BE4)], RW[b], SEM[b])
        pltpu.async_copy(coli.at[pl.ds(base, BE4)], CW[b], SEM[b])
        pltpu.async_copy(mflat.at[pl.ds(base, BE4)], MW[b], SEM[b])

    def compute(ci, b):
        base = base0 + ci * BE4
        pltpu.make_async_copy(rowi.at[pl.ds(base, BE4)], RW[b], SEM[b]).wait()
        pltpu.make_async_copy(coli.at[pl.ds(base, BE4)], CW[b], SEM[b]).wait()
        pltpu.make_async_copy(mflat.at[pl.ds(base, BE4)], MW[b], SEM[b]).wait()
        rw, cw, mw = RW[b], CW[b], MW[b]

        def grp(g, c):
            sl = pl.ds(g * LANES, LANES)
            r = rw[sl]
            r3 = r * 3
            c3 = cw[sl] * 3
            m16 = mw[sl]
            dx = _cgather(coordv, r3) - _cgather(coordv, c3)
            dy = _cgather(coordv, r3 + 1) - _cgather(coordv, c3 + 1)
            dz = _cgather(coordv, r3 + 2) - _cgather(coordv, c3 + 2)
            tx = jnp.clip(m16 * dx, -100.0, 100.0)
            ty = jnp.clip(m16 * dy, -100.0, 100.0)
            tz = jnp.clip(m16 * dz, -100.0, 100.0)
            arow = lax.shift_right_logical(r, 5)
            acol = (r & 31) * 4
            plsc.addupdate_scatter(acc, [arow, acol], tx)
            plsc.addupdate_scatter(acc, [arow, acol + 1], ty)
            plsc.addupdate_scatter(acc, [arow, acol + 2], tz)
            plsc.addupdate_scatter(acc, [arow, acol + 3], ones16)
            return c

        lax.fori_loop(0, BE4 // LANES, grp, 0)

    io(0, 0)

    def pipe(j, carry):
        c0 = j * 2
        io(c0 + 1, 1)
        compute(c0, 0)

        @pl.when(c0 + 2 < NCH4)
        def _():
            io(c0 + 2, 0)

        compute(c0 + 1, 1)
        return carry

    lax.fori_loop(0, NCH4 // 2, pipe, 0)
    compute(NCH4 - 1, 0)

    pltpu.sync_copy(acc, sstg.at[pl.ds(sid * 320, 320)])
    plsc.subcore_barrier()

    @pl.when(sid < 10)
    def _():
        def zr_body(t, c):
            i = t // (D // LANES)
            k = lax.rem(t, D // LANES)
            redbuf[i, pl.ds(k * LANES, LANES)] = zero16
            return c

        lax.fori_loop(0, 32 * (D // LANES), zr_body, 0)

        def red(p, c):
            pltpu.sync_copy(sstg.at[pl.ds(p * 320 + sid * 32, 32)], pbuf)

            def addp(t, cc):
                i = t // (D // LANES)
                k = lax.rem(t, D // LANES)
                slk = pl.ds(k * LANES, LANES)
                redbuf[i, slk] = redbuf[i, slk] + pbuf[i, slk]
                return cc

            lax.fori_loop(0, 32 * (D // LANES), addp, 0)
            return c

        lax.fori_loop(0, NS, red, 0)

        def batch(bi, c):
            def ex(g, cc):
                nloc = lax.iota(jnp.int32, LANES) + g * LANES
                srow = bi + lax.shift_right_logical(nloc, 5)
                scol = (nloc & 31) * 4
                for k in range(4):
                    val = plsc.load_gather(redbuf, [srow, scol + k])
                    plsc.store_scatter(ebuf, [nloc, jnp.full((LANES,), k, jnp.int32)], val)
                return cc

            lax.fori_loop(0, 32 // LANES, ex, 0)
            nb = sid * 1024 + bi * 32
            pltpu.sync_copy(ebuf, sums_out.at[cid].at[pl.ds(nb, 32)])
            return c

        lax.fori_loop(0, 32, batch, 0)


def _k4b(mflat, rowi, coli, coordf):
    return pl.kernel(
        _k4b_body,
        out_type=jax.ShapeDtypeStruct((NC, N_PAD, D), jnp.float32),
        mesh=_mesh(),
        compiler_params=pltpu.CompilerParams(needs_layout_passes=False),
        scratch_types=[
            pltpu.VMEM((BE4,), jnp.int32),
            pltpu.VMEM((BE4,), jnp.int32),
            pltpu.VMEM((BE4,), jnp.int32),
            pltpu.VMEM((BE4,), jnp.int32),
            pltpu.VMEM((BE4,), jnp.float32),
            pltpu.VMEM((BE4,), jnp.float32),
            pltpu.VMEM((320, D), jnp.float32),
            pltpu.VMEM((32, D), jnp.float32),
            pltpu.VMEM((32, D), jnp.float32),
            pltpu.VMEM((32, D), jnp.float32),
            pltpu.VMEM((CROWS, 128), jnp.float32),
            pltpu.VMEM_SHARED((NS * 320, D), jnp.float32),
            pltpu.SemaphoreType.DMA,
            pltpu.SemaphoreType.DMA,
        ],
    )(mflat, rowi, coli, coordf)


def _k5_body(h_ref, agg_ref, sums_ref, coord_ref, w1h_ref, w1a_ref, bn1_ref,
             wn2_ref, bn2_ref, hout_ref, cout_ref):
    agg = agg_ref[0] + agg_ref[1]
    hb = h_ref[...]
    t = jnp.maximum(
        _bdot(hb, w1h_ref[...]) + _bdot(agg, w1a_ref[...]) + bn1_ref[...], 0.0)
    hout_ref[...] = hb + _bdot(t, wn2_ref[...]) + bn2_ref[...]
    s = sums_ref[0] + sums_ref[1]
    cnt = jnp.maximum(s[:, 3:4], 1.0)
    cout_ref[...] = coord_ref[...] + s[:, 0:3] / cnt


def _k5(h, aggp, sumsp, coord, w1h, w1a, bn1, wn2, bn2):
    return pl.pallas_call(
        _k5_body,
        grid=(NNB5,),
        in_specs=[
            pl.BlockSpec((NB5, D), lambda i: (i, 0)),
            pl.BlockSpec((NC, NB5, H), lambda i: (0, i, 0)),
            pl.BlockSpec((NC, NB5, D), lambda i: (0, i, 0)),
            pl.BlockSpec((NB5, 3), lambda i: (i, 0)),
            pl.BlockSpec((D, H), lambda i: (0, 0)),
            pl.BlockSpec((H, H), lambda i: (0, 0)),
            pl.BlockSpec((1, H), lambda i: (0, 0)),
            pl.BlockSpec((H, D), lambda i: (0, 0)),
            pl.BlockSpec((1, D), lambda i: (0, 0)),
        ],
        out_specs=[
            pl.BlockSpec((NB5, D), lambda i: (i, 0)),
            pl.BlockSpec((NB5, 3), lambda i: (i, 0)),
        ],
        out_shape=[
            jax.ShapeDtypeStruct((N, D), jnp.float32),
            jax.ShapeDtypeStruct((N, 3), jnp.float32),
        ],
    )(h, aggp, sumsp, coord, w1h, w1a, bn1, wn2, bn2)


def kernel(h, edge_index, coord, edge_attr, We1, be1, We2, be2,
           Wn1, bn1, Wn2, bn2, Wc1, bc1, Wc2):
    row = edge_index[0]
    col = edge_index[1]
    wa = We1[0:D]
    wb = We1[D:2 * D]
    wr = We1[2 * D:2 * D + 1]
    wc = We1[2 * D + 1:]
    coordf = jnp.pad(coord.reshape(-1), (0, CROWS * 128 - 3 * N)).reshape(CROWS, 128)

    hwa, hwb = _k1(h, wa, wb)
    g, rad = _k2(hwa, hwb, row, col, coordf)
    rad2 = rad.reshape(NEB, 1, EB)
    ef, m2 = _k3(g, edge_attr, rad2, wc, wr, be1.reshape(1, H), We2,
                 be2.reshape(1, H), Wc1, bc1.reshape(1, H), Wc2)
    mflat = m2.reshape(-1)
    aggp = _k4a(ef, row)
    sumsp = _k4b(mflat, row, col, coordf)
    hout, cpart = _k5(h, aggp, sumsp, coord, Wn1[:D], Wn1[D:],
                      bn1.reshape(1, H), Wn2, bn2.reshape(1, D))
    return hout, cpart[:, :, None]


# speedup vs baseline: 7.9099x; 1.0391x over previous
import jax
import jax.numpy as jnp
from jax import lax
from jax.experimental import pallas as pl
from jax.experimental.pallas import tpu as pltpu
from jax.experimental.pallas import tpu_sc as plsc

N = 10000
E = 320000
D = 128
H = 128

NC = 2
NS = 16
NW = NC * NS
EW = E // NW
BE = 80
NCH = EW // BE
LANES = 16

EB = 8000
NEB = E // EB

NB = 1000
NNB = N // NB
NB5 = 1024
NNB5 = 10


CROWS = 240


def _bdot(a, b):
    return jnp.dot(a.astype(jnp.bfloat16), b.astype(jnp.bfloat16),
                   preferred_element_type=jnp.float32)


def _cgather(coordv, flat_idx):
    return plsc.load_gather(
        coordv, [lax.shift_right_logical(flat_idx, 7), flat_idx & 127])


def _mesh():
    return plsc.VectorSubcoreMesh(
        core_axis_name="c", subcore_axis_name="s", num_cores=NC, num_subcores=NS
    )


def _k1_body(h_ref, wa_ref, wb_ref, a_ref, b_ref):
    hb = h_ref[...]
    a_ref[...] = _bdot(hb, wa_ref[...])
    b_ref[...] = _bdot(hb, wb_ref[...])


def _k1(h, wa, wb):
    return pl.pallas_call(
        _k1_body,
        grid=(NNB,),
        in_specs=[
            pl.BlockSpec((NB, D), lambda i: (i, 0)),
            pl.BlockSpec((D, D), lambda i: (0, 0)),
            pl.BlockSpec((D, D), lambda i: (0, 0)),
        ],
        out_specs=[
            pl.BlockSpec((NB, D), lambda i: (i, 0)),
            pl.BlockSpec((NB, D), lambda i: (i, 0)),
        ],
        out_shape=[
            jax.ShapeDtypeStruct((N, D), jnp.float32),
            jax.ShapeDtypeStruct((N, D), jnp.float32),
        ],
    )(h, wa, wb)


def _k2_body(hwa, hwb, rowi, coli, coordf, g_out, rad_out,
             rowv0, rowv1, colv0, colv1, bufa0, bufa1, bufb0, bufb1,
             radv, coordv, sema0, sema1, semb0, semb1):
    cid = lax.axis_index("c")
    sid = lax.axis_index("s")
    wid = sid * NC + cid
    base0 = wid * EW
    pltpu.sync_copy(coordf, coordv)

    ROWV = [rowv0, rowv1]
    COLV = [colv0, colv1]
    BUFA = [bufa0, bufa1]
    BUFB = [bufb0, bufb1]
    SEMA = [sema0, sema1]
    SEMB = [semb0, semb1]

    def io(ci, b):
        base = base0 + ci * BE
        pltpu.sync_copy(rowi.at[pl.ds(base, BE)], ROWV[b])
        pltpu.sync_copy(coli.at[pl.ds(base, BE)], COLV[b])
        pltpu.async_copy(hwa.at[ROWV[b]], BUFA[b], SEMA[b])
        pltpu.async_copy(hwb.at[COLV[b]], BUFB[b], SEMB[b])

    def compute(ci, b):
        base = base0 + ci * BE
        pltpu.make_async_copy(hwa.at[ROWV[b]], BUFA[b], SEMA[b]).wait()
        pltpu.make_async_copy(hwb.at[COLV[b]], BUFB[b], SEMB[b]).wait()
        ba, bb = BUFA[b], BUFB[b]

        def add_body(i, c):
            for k in range(D // LANES):
                sl = pl.ds(k * LANES, LANES)
                ba[i, sl] = ba[i, sl] + bb[i, sl]
            return c

        lax.fori_loop(0, BE, add_body, 0)

        rv, cv = ROWV[b], COLV[b]

        def rad_body(g, c):
            sl = pl.ds(g * LANES, LANES)
            r3 = rv[sl] * 3
            c3 = cv[sl] * 3
            dx = _cgather(coordv, r3) - _cgather(coordv, c3)
            dy = _cgather(coordv, r3 + 1) - _cgather(coordv, c3 + 1)
            dz = _cgather(coordv, r3 + 2) - _cgather(coordv, c3 + 2)
            radv[sl] = dx * dx + dy * dy + dz * dz
            return c

        lax.fori_loop(0, BE // LANES, rad_body, 0)
        pltpu.sync_copy(ba, g_out.at[pl.ds(base, BE)])
        pltpu.sync_copy(radv, rad_out.at[pl.ds(base, BE)])

    io(0, 0)

    def pipe(j, carry):
        c0 = j * 2
        io(c0 + 1, 1)
        compute(c0, 0)
        io(c0 + 2, 0)
        compute(c0 + 1, 1)
        return carry

    lax.fori_loop(0, (NCH - 1) // 2, pipe, 0)
    compute(NCH - 1, 0)


def _k2(hwa, hwb, rowi, coli, coordf):
    return pl.kernel(
        _k2_body,
        out_type=[
            jax.ShapeDtypeStruct((E, D), jnp.float32),
            jax.ShapeDtypeStruct((E,), jnp.float32),
        ],
        mesh=_mesh(),
        compiler_params=pltpu.CompilerParams(needs_layout_passes=False),
        scratch_types=[
            pltpu.VMEM((BE,), jnp.int32),
            pltpu.VMEM((BE,), jnp.int32),
            pltpu.VMEM((BE,), jnp.int32),
            pltpu.VMEM((BE,), jnp.int32),
            pltpu.VMEM((BE, D), jnp.float32),
            pltpu.VMEM((BE, D), jnp.float32),
            pltpu.VMEM((BE, D), jnp.float32),
            pltpu.VMEM((BE, D), jnp.float32),
            pltpu.VMEM((BE,), jnp.float32),
            pltpu.VMEM((CROWS, 128), jnp.float32),
            pltpu.SemaphoreType.DMA,
            pltpu.SemaphoreType.DMA,
            pltpu.SemaphoreType.DMA,
            pltpu.SemaphoreType.DMA,
        ],
    )(hwa, hwb, rowi, coli, coordf)


def _k3_body(g_ref, ea_ref, rad_ref, wc_ref, wr_ref, be1_ref, w2_ref, be2_ref,
             wc1_ref, bc1_ref, wc2_ref, ef_ref, m_ref):
    rad_row = rad_ref[0]
    radp = lax.dot_general(rad_row, wr_ref[...], (((0,), (0,)), ((), ())),
                           preferred_element_type=jnp.float32)
    pre = (g_ref[...] + _bdot(ea_ref[...], wc_ref[...]) + radp + be1_ref[...])
    hdn = jnp.maximum(pre, 0.0)
    ef = jnp.maximum(_bdot(hdn, w2_ref[...]) + be2_ref[...], 0.0)
    ef_ref[...] = ef
    t = jnp.maximum(_bdot(ef, wc1_ref[...]) + bc1_ref[...], 0.0)
    m_ref[0] = lax.dot_general(wc2_ref[...], t, (((0,), (1,)), ((), ())),
                               preferred_element_type=jnp.float32)


def _k3(g, ea, rad2, wc, wr, be1, w2, be2, wc1, bc1, wc2):
    return pl.pallas_call(
        _k3_body,
        grid=(NEB,),
        in_specs=[
            pl.BlockSpec((EB, D), lambda i: (i, 0)),
            pl.BlockSpec((EB, H), lambda i: (i, 0)),
            pl.BlockSpec((1, 1, EB), lambda i: (i, 0, 0)),
            pl.BlockSpec((H, H), lambda i: (0, 0)),
            pl.BlockSpec((1, H), lambda i: (0, 0)),
            pl.BlockSpec((1, H), lambda i: (0, 0)),
            pl.BlockSpec((H, H), lambda i: (0, 0)),
            pl.BlockSpec((1, H), lambda i: (0, 0)),
            pl.BlockSpec((H, H), lambda i: (0, 0)),
            pl.BlockSpec((1, H), lambda i: (0, 0)),
            pl.BlockSpec((H, 1), lambda i: (0, 0)),
        ],
        out_specs=[
            pl.BlockSpec((EB, H), lambda i: (i, 0)),
            pl.BlockSpec((1, 1, EB), lambda i: (i, 0, 0)),
        ],
        out_shape=[
            jax.ShapeDtypeStruct((E, H), jnp.float32),
            jax.ShapeDtypeStruct((NEB, 1, EB), jnp.float32),
        ],
    )(g, ea, rad2, wc, wr, be1, w2, be2, wc1, bc1, wc2)


N_PAD = 10240
NROWS_SUB = N_PAD // NS
ZB = 128
NSR = N_PAD // 8


def _k4a_body(ef, rowi, agg_out, rw0, rw1, ef0, ef1, zbuf, aggS, sem0, sem1):
    cid = lax.axis_index("c")
    sid = lax.axis_index("s")
    wid = sid * NC + cid
    base0 = wid * EW

    RW = [rw0, rw1]
    EFV = [ef0, ef1]
    SEM = [sem0, sem1]

    zero16 = jnp.zeros((LANES,), jnp.float32)

    def zb_body(t, c):
        i = t // (D // LANES)
        k = lax.rem(t, D // LANES)
        zbuf[i, pl.ds(k * LANES, LANES)] = zero16
        return c

    lax.fori_loop(0, ZB * (D // LANES), zb_body, 0)

    def zc_body(j, c):
        pltpu.sync_copy(zbuf, aggS.at[pl.ds(sid * NROWS_SUB + j * ZB, ZB)])
        return c

    lax.fori_loop(0, NROWS_SUB // ZB, zc_body, 0)
    plsc.subcore_barrier()

    def io(ci, b):
        base = base0 + ci * BE
        pltpu.async_copy(rowi.at[pl.ds(base, BE)], RW[b], SEM[b])
        pltpu.async_copy(ef.at[pl.ds(base, BE)], EFV[b], SEM[b])

    def scat(ci, b):
        base = base0 + ci * BE
        pltpu.make_async_copy(rowi.at[pl.ds(base, BE)], RW[b], SEM[b]).wait()
        pltpu.make_async_copy(ef.at[pl.ds(base, BE)], EFV[b], SEM[b]).wait()
        pltpu.sync_copy(EFV[b], aggS.at[RW[b]], add=True)

    io(0, 0)

    def pipe(j, carry):
        c0 = j * 2
        io(c0 + 1, 1)
        scat(c0, 0)
        io(c0 + 2, 0)
        scat(c0 + 1, 1)
        return carry

    lax.fori_loop(0, (NCH - 1) // 2, pipe, 0)
    scat(NCH - 1, 0)
    plsc.subcore_barrier()

    def dump(j, c):
        sl = pl.ds(sid * NROWS_SUB + j * ZB, ZB)
        pltpu.sync_copy(aggS.at[sl], agg_out.at[cid].at[sl])
        return c

    lax.fori_loop(0, NROWS_SUB // ZB, dump, 0)


def _k4a(ef, rowi):
    return pl.kernel(
        _k4a_body,
        out_type=jax.ShapeDtypeStruct((NC, N_PAD, H), jnp.float32),
        mesh=_mesh(),
        compiler_params=pltpu.CompilerParams(needs_layout_passes=False),
        scratch_types=[
            pltpu.VMEM((BE,), jnp.int32),
            pltpu.VMEM((BE,), jnp.int32),
            pltpu.VMEM((BE, H), jnp.float32),
            pltpu.VMEM((BE, H), jnp.float32),
            pltpu.VMEM((ZB, D), jnp.float32),
            pltpu.VMEM_SHARED((N_PAD, H), jnp.float32),
            pltpu.SemaphoreType.DMA,
            pltpu.SemaphoreType.DMA,
        ],
    )(ef, rowi)


BE4 = 400


def _k4b_body(mflat, rowi, coli, coordf, sums_out,
              rw0, rw1, cw0, cw1, mw0, mw1, acc, pbuf, redbuf, ebuf, coordv,
              sstg, sem0, sem1):
    cid = lax.axis_index("c")
    sid = lax.axis_index("s")
    wid = sid * NC + cid
    base0 = wid * EW
    pltpu.sync_copy(coordf, coordv)

    RW = [rw0, rw1]
    CW = [cw0, cw1]
    MW = [mw0, mw1]
    SEM = [sem0, sem1]
    NCH4 = EW // BE4

    zero16 = jnp.zeros((LANES,), jnp.float32)
    ones16 = jnp.ones((LANES,), jnp.float32)

    def za_body(t, c):
        i = t // (D // LANES)
        k = lax.rem(t, D // LANES)
        acc[i, pl.ds(k * LANES, LANES)] = zero16
        return c

    lax.fori_loop(0, 320 * (D // LANES), za_body, 0)

    def ze_body(t, c):
        i = t // (D // LANES)
        k = lax.rem(t, D // LANES)
        ebuf[i, pl.ds(k * LANES, LANES)] = zero16
        return c

    lax.fori_loop(0, 32 * (D // LANES), ze_body, 0)

    def io(ci, b):
        base = base0 + ci * BE4
        pltpu.async_copy(rowi.at[pl.ds(base, BE4)], RW[b], SEM[b])
        pltpu.async_copy(coli.at[pl.ds(base, BE4)], CW[b], SEM[b])
        pltpu.async_copy(mflat.at[pl.ds(base, BE4)], MW[b], SEM[b])

    def compute(ci, b):
        base = base0 + ci * BE4
        pltpu.make_async_copy(rowi.at[pl.ds(base, BE4)], RW[b], SEM[b]).wait()
        pltpu.make_async_copy(coli.at[pl.ds(base, BE4)], CW[b], SEM[b]).wait()
        pltpu.make_async_copy(mflat.at[pl.ds(base, BE4)], MW[b], SEM[b]).wait()
        rw, cw, mw = RW[b], CW[b], MW[b]

        def grp(g, c):
            sl = pl.ds(g * LANES, LANES)
            r = rw[sl]
            r3 = r * 3
            c3 = cw[sl] * 3
            m16 = mw[sl]
            dx = _cgather(coordv, r3) - _cgather(coordv, c3)
            dy = _cgather(coordv, r3 + 1) - _cgather(coordv, c3 + 1)
            dz = _cgather(coordv, r3 + 2) - _cgather(coordv, c3 + 2)
            tx = jnp.clip(m16 * dx, -100.0, 100.0)
            ty = jnp.clip(m16 * dy, -100.0, 100.0)
            tz = jnp.clip(m16 * dz, -100.0, 100.0)
            arow = lax.shift_right_logical(r, 5)
            acol = (r & 31) * 4
            plsc.addupdate_scatter(acc, [arow, acol], tx)
            plsc.addupdate_scatter(acc, [arow, acol + 1], ty)
            plsc.addupdate_scatter(acc, [arow, acol + 2], tz)
            plsc.addupdate_scatter(acc, [arow, acol + 3], ones16)
            return c

        lax.fori_loop(0, BE4 // LANES, grp, 0)

    io(0, 0)

    def pipe(j, carry):
        c0 = j * 2
        io(c0 + 1, 1)
        compute(c0, 0)

        @pl.when(c0 + 2 < NCH4)
        def _():
            io(c0 + 2, 0)

        compute(c0 + 1, 1)
        return carry

    lax.fori_loop(0, NCH4 // 2, pipe, 0)
    compute(NCH4 - 1, 0)

    pltpu.sync_copy(acc, sstg.at[pl.ds(sid * 320, 320)])
    plsc.subcore_barrier()

    @pl.when(sid < 10)
    def _():
        def zr_body(t, c):
            i = t // (D // LANES)
            k = lax.rem(t, D // LANES)
            redbuf[i, pl.ds(k * LANES, LANES)] = zero16
            return c

        lax.fori_loop(0, 32 * (D // LANES), zr_body, 0)

        def red(p, c):
            pltpu.sync_copy(sstg.at[pl.ds(p * 320 + sid * 32, 32)], pbuf)

            def addp(t, cc):
                i = t // (D // LANES)
                k = lax.rem(t, D // LANES)
                slk = pl.ds(k * LANES, LANES)
                redbuf[i, slk] = redbuf[i, slk] + pbuf[i, slk]
                return cc

            lax.fori_loop(0, 32 * (D // LANES), addp, 0)
            return c

        lax.fori_loop(0, NS, red, 0)

        def batch(bi, c):
            def ex(g, cc):
                nloc = lax.iota(jnp.int32, LANES) + g * LANES
                srow = bi + lax.shift_right_logical(nloc, 5)
                scol = (nloc & 31) * 4
                for k in range(4):
                    val = plsc.load_gather(redbuf, [srow, scol + k])
                    plsc.store_scatter(ebuf, [nloc, jnp.full((LANES,), k, jnp.int32)], val)
                return cc

            lax.fori_loop(0, 32 // LANES, ex, 0)
            nb = sid * 1024 + bi * 32
            pltpu.sync_copy(ebuf, sums_out.at[cid].at[pl.ds(nb, 32)])
            return c

        lax.fori_loop(0, 32, batch, 0)


def _k4b(mflat, rowi, coli, coordf):
    return pl.kernel(
        _k4b_body,
        out_type=jax.ShapeDtypeStruct((NC, N_PAD, D), jnp.float32),
        mesh=_mesh(),
        compiler_params=pltpu.CompilerParams(needs_layout_passes=False),
        scratch_types=[
            pltpu.VMEM((BE4,), jnp.int32),
            pltpu.VMEM((BE4,), jnp.int32),
            pltpu.VMEM((BE4,), jnp.int32),
            pltpu.VMEM((BE4,), jnp.int32),
            pltpu.VMEM((BE4,), jnp.float32),
            pltpu.VMEM((BE4,), jnp.float32),
            pltpu.VMEM((320, D), jnp.float32),
            pltpu.VMEM((32, D), jnp.float32),
            pltpu.VMEM((32, D), jnp.float32),
            pltpu.VMEM((32, D), jnp.float32),
            pltpu.VMEM((CROWS, 128), jnp.float32),
            pltpu.VMEM_SHARED((NS * 320, D), jnp.float32),
            pltpu.SemaphoreType.DMA,
            pltpu.SemaphoreType.DMA,
        ],
    )(mflat, rowi, coli, coordf)


def _k5_body(h_ref, agg_ref, sums_ref, coord_ref, w1h_ref, w1a_ref, bn1_ref,
             wn2_ref, bn2_ref, hout_ref, cout_ref):
    agg = agg_ref[0] + agg_ref[1]
    hb = h_ref[...]
    t = jnp.maximum(
        _bdot(hb, w1h_ref[...]) + _bdot(agg, w1a_ref[...]) + bn1_ref[...], 0.0)
    hout_ref[...] = hb + _bdot(t, wn2_ref[...]) + bn2_ref[...]
    s = sums_ref[0] + sums_ref[1]
    cnt = jnp.maximum(s[:, 3:4], 1.0)
    cout_ref[...] = coord_ref[...] + s[:, 0:3] / cnt


def _k5(h, aggp, sumsp, coord, w1h, w1a, bn1, wn2, bn2):
    return pl.pallas_call(
        _k5_body,
        grid=(NNB5,),
        in_specs=[
            pl.BlockSpec((NB5, D), lambda i: (i, 0)),
            pl.BlockSpec((NC, NB5, H), lambda i: (0, i, 0)),
            pl.BlockSpec((NC, NB5, D), lambda i: (0, i, 0)),
            pl.BlockSpec((NB5, 3), lambda i: (i, 0)),
            pl.BlockSpec((D, H), lambda i: (0, 0)),
            pl.BlockSpec((H, H), lambda i: (0, 0)),
            pl.BlockSpec((1, H), lambda i: (0, 0)),
            pl.BlockSpec((H, D), lambda i: (0, 0)),
            pl.BlockSpec((1, D), lambda i: (0, 0)),
        ],
        out_specs=[
            pl.BlockSpec((NB5, D), lambda i: (i, 0)),
            pl.BlockSpec((NB5, 3), lambda i: (i, 0)),
        ],
        out_shape=[
            jax.ShapeDtypeStruct((N, D), jnp.float32),
            jax.ShapeDtypeStruct((N, 3), jnp.float32),
        ],
    )(h, aggp, sumsp, coord, w1h, w1a, bn1, wn2, bn2)


def kernel(h, edge_index, coord, edge_attr, We1, be1, We2, be2,
           Wn1, bn1, Wn2, bn2, Wc1, bc1, Wc2):
    row = edge_index[0]
    col = edge_index[1]
    wa = We1[0:D]
    wb = We1[D:2 * D]
    wr = We1[2 * D:2 * D + 1]
    wc = We1[2 * D + 1:]
    coordf = jnp.pad(coord.reshape(-1), (0, CROWS * 128 - 3 * N)).reshape(CROWS, 128)

    hwa, hwb = _k1(h, wa, wb)
    g, rad = _k2(hwa, hwb, row, col, coordf)
    rad2 = rad.reshape(NEB, 1, EB)
    ef, m2 = _k3(g, edge_attr, rad2, wc, wr, be1.reshape(1, H), We2,
                 be2.reshape(1, H), Wc1, bc1.reshape(1, H), Wc2)
    mflat = m2.reshape(-1)
    aggp = _k4a(ef, row)
    sumsp = _k4b(mflat, row, col, coordf)
    hout, cpart = _k5(h, aggp, sumsp, coord, Wn1[:D], Wn1[D:],
                      bn1.reshape(1, H), Wn2, bn2.reshape(1, D))
    return hout, cpart[:, :, None]


# speedup vs baseline: 7.9439x; 1.0043x over previous
import jax
import jax.numpy as jnp
from jax import lax
from jax.experimental import pallas as pl
from jax.experimental.pallas import tpu as pltpu
from jax.experimental.pallas import tpu_sc as plsc

N = 10000
E = 320000
D = 128
H = 128

NC = 2
NS = 16
NW = NC * NS
EW = E // NW
BE = 80
NCH = EW // BE
LANES = 16

EB = 16000
NEB = E // EB

NB = 1000
NNB = N // NB
NB5 = 1024
NNB5 = 10


CROWS = 240


def _bdot(a, b):
    return jnp.dot(a.astype(jnp.bfloat16), b.astype(jnp.bfloat16),
                   preferred_element_type=jnp.float32)


def _cgather(coordv, flat_idx):
    return plsc.load_gather(
        coordv, [lax.shift_right_logical(flat_idx, 7), flat_idx & 127])


def _mesh():
    return plsc.VectorSubcoreMesh(
        core_axis_name="c", subcore_axis_name="s", num_cores=NC, num_subcores=NS
    )


def _k1_body(h_ref, wa_ref, wb_ref, a_ref, b_ref):
    hb = h_ref[...]
    a_ref[...] = _bdot(hb, wa_ref[...])
    b_ref[...] = _bdot(hb, wb_ref[...])


def _k1(h, wa, wb):
    return pl.pallas_call(
        _k1_body,
        grid=(NNB,),
        in_specs=[
            pl.BlockSpec((NB, D), lambda i: (i, 0)),
            pl.BlockSpec((D, D), lambda i: (0, 0)),
            pl.BlockSpec((D, D), lambda i: (0, 0)),
        ],
        out_specs=[
            pl.BlockSpec((NB, D), lambda i: (i, 0)),
            pl.BlockSpec((NB, D), lambda i: (i, 0)),
        ],
        out_shape=[
            jax.ShapeDtypeStruct((N, D), jnp.float32),
            jax.ShapeDtypeStruct((N, D), jnp.float32),
        ],
    )(h, wa, wb)


def _k2_body(hwa, hwb, rowi, coli, coordf, g_out, rad_out,
             rowv0, rowv1, colv0, colv1, bufa0, bufa1, bufb0, bufb1,
             radv, coordv, sema0, sema1, semb0, semb1):
    cid = lax.axis_index("c")
    sid = lax.axis_index("s")
    wid = sid * NC + cid
    base0 = wid * EW
    pltpu.sync_copy(coordf, coordv)

    ROWV = [rowv0, rowv1]
    COLV = [colv0, colv1]
    BUFA = [bufa0, bufa1]
    BUFB = [bufb0, bufb1]
    SEMA = [sema0, sema1]
    SEMB = [semb0, semb1]

    def io(ci, b):
        base = base0 + ci * BE
        pltpu.sync_copy(rowi.at[pl.ds(base, BE)], ROWV[b])
        pltpu.sync_copy(coli.at[pl.ds(base, BE)], COLV[b])
        pltpu.async_copy(hwa.at[ROWV[b]], BUFA[b], SEMA[b])
        pltpu.async_copy(hwb.at[COLV[b]], BUFB[b], SEMB[b])

    def compute(ci, b):
        base = base0 + ci * BE
        pltpu.make_async_copy(hwa.at[ROWV[b]], BUFA[b], SEMA[b]).wait()
        pltpu.make_async_copy(hwb.at[COLV[b]], BUFB[b], SEMB[b]).wait()
        ba, bb = BUFA[b], BUFB[b]

        def add_body(i, c):
            for k in range(D // LANES):
                sl = pl.ds(k * LANES, LANES)
                ba[i, sl] = ba[i, sl] + bb[i, sl]
            return c

        lax.fori_loop(0, BE, add_body, 0)

        rv, cv = ROWV[b], COLV[b]

        def rad_body(g, c):
            sl = pl.ds(g * LANES, LANES)
            r3 = rv[sl] * 3
            c3 = cv[sl] * 3
            dx = _cgather(coordv, r3) - _cgather(coordv, c3)
            dy = _cgather(coordv, r3 + 1) - _cgather(coordv, c3 + 1)
            dz = _cgather(coordv, r3 + 2) - _cgather(coordv, c3 + 2)
            radv[sl] = dx * dx + dy * dy + dz * dz
            return c

        lax.fori_loop(0, BE // LANES, rad_body, 0)
        pltpu.sync_copy(ba, g_out.at[pl.ds(base, BE)])
        pltpu.sync_copy(radv, rad_out.at[pl.ds(base, BE)])

    io(0, 0)

    def pipe(j, carry):
        c0 = j * 2
        io(c0 + 1, 1)
        compute(c0, 0)
        io(c0 + 2, 0)
        compute(c0 + 1, 1)
        return carry

    lax.fori_loop(0, (NCH - 1) // 2, pipe, 0)
    compute(NCH - 1, 0)


def _k2(hwa, hwb, rowi, coli, coordf):
    return pl.kernel(
        _k2_body,
        out_type=[
            jax.ShapeDtypeStruct((E, D), jnp.float32),
            jax.ShapeDtypeStruct((E,), jnp.float32),
        ],
        mesh=_mesh(),
        compiler_params=pltpu.CompilerParams(needs_layout_passes=False),
        scratch_types=[
            pltpu.VMEM((BE,), jnp.int32),
            pltpu.VMEM((BE,), jnp.int32),
            pltpu.VMEM((BE,), jnp.int32),
            pltpu.VMEM((BE,), jnp.int32),
            pltpu.VMEM((BE, D), jnp.float32),
            pltpu.VMEM((BE, D), jnp.float32),
            pltpu.VMEM((BE, D), jnp.float32),
            pltpu.VMEM((BE, D), jnp.float32),
            pltpu.VMEM((BE,), jnp.float32),
            pltpu.VMEM((CROWS, 128), jnp.float32),
            pltpu.SemaphoreType.DMA,
            pltpu.SemaphoreType.DMA,
            pltpu.SemaphoreType.DMA,
            pltpu.SemaphoreType.DMA,
        ],
    )(hwa, hwb, rowi, coli, coordf)


def _k3_body(g_ref, ea_ref, rad_ref, wc_ref, wr_ref, be1_ref, w2_ref, be2_ref,
             wc1_ref, bc1_ref, wc2_ref, ef_ref, m_ref):
    rad_row = rad_ref[0]
    radp = lax.dot_general(rad_row, wr_ref[...], (((0,), (0,)), ((), ())),
                           preferred_element_type=jnp.float32)
    pre = (g_ref[...] + _bdot(ea_ref[...], wc_ref[...]) + radp + be1_ref[...])
    hdn = jnp.maximum(pre, 0.0)
    ef = jnp.maximum(_bdot(hdn, w2_ref[...]) + be2_ref[...], 0.0)
    ef_ref[...] = ef
    t = jnp.maximum(_bdot(ef, wc1_ref[...]) + bc1_ref[...], 0.0)
    m_ref[0] = lax.dot_general(wc2_ref[...], t, (((0,), (1,)), ((), ())),
                               preferred_element_type=jnp.float32)


def _k3(g, ea, rad2, wc, wr, be1, w2, be2, wc1, bc1, wc2):
    return pl.pallas_call(
        _k3_body,
        grid=(NEB,),
        in_specs=[
            pl.BlockSpec((EB, D), lambda i: (i, 0)),
            pl.BlockSpec((EB, H), lambda i: (i, 0)),
            pl.BlockSpec((1, 1, EB), lambda i: (i, 0, 0)),
            pl.BlockSpec((H, H), lambda i: (0, 0)),
            pl.BlockSpec((1, H), lambda i: (0, 0)),
            pl.BlockSpec((1, H), lambda i: (0, 0)),
            pl.BlockSpec((H, H), lambda i: (0, 0)),
            pl.BlockSpec((1, H), lambda i: (0, 0)),
            pl.BlockSpec((H, H), lambda i: (0, 0)),
            pl.BlockSpec((1, H), lambda i: (0, 0)),
            pl.BlockSpec((H, 1), lambda i: (0, 0)),
        ],
        out_specs=[
            pl.BlockSpec((EB, H), lambda i: (i, 0)),
            pl.BlockSpec((1, 1, EB), lambda i: (i, 0, 0)),
        ],
        out_shape=[
            jax.ShapeDtypeStruct((E, H), jnp.float32),
            jax.ShapeDtypeStruct((NEB, 1, EB), jnp.float32),
        ],
    )(g, ea, rad2, wc, wr, be1, w2, be2, wc1, bc1, wc2)


N_PAD = 10240
NROWS_SUB = N_PAD // NS
ZB = 128
NSR = N_PAD // 8


def _k4a_body(ef, rowi, agg_out, rw0, rw1, ef0, ef1, zbuf, aggS, sem0, sem1):
    cid = lax.axis_index("c")
    sid = lax.axis_index("s")
    wid = sid * NC + cid
    base0 = wid * EW

    RW = [rw0, rw1]
    EFV = [ef0, ef1]
    SEM = [sem0, sem1]

    zero16 = jnp.zeros((LANES,), jnp.float32)

    def zb_body(t, c):
        i = t // (D // LANES)
        k = lax.rem(t, D // LANES)
        zbuf[i, pl.ds(k * LANES, LANES)] = zero16
        return c

    lax.fori_loop(0, ZB * (D // LANES), zb_body, 0)

    def zc_body(j, c):
        pltpu.sync_copy(zbuf, aggS.at[pl.ds(sid * NROWS_SUB + j * ZB, ZB)])
        return c

    lax.fori_loop(0, NROWS_SUB // ZB, zc_body, 0)
    plsc.subcore_barrier()

    def io(ci, b):
        base = base0 + ci * BE
        pltpu.async_copy(rowi.at[pl.ds(base, BE)], RW[b], SEM[b])
        pltpu.async_copy(ef.at[pl.ds(base, BE)], EFV[b], SEM[b])

    def scat(ci, b):
        base = base0 + ci * BE
        pltpu.make_async_copy(rowi.at[pl.ds(base, BE)], RW[b], SEM[b]).wait()
        pltpu.make_async_copy(ef.at[pl.ds(base, BE)], EFV[b], SEM[b]).wait()
        pltpu.sync_copy(EFV[b], aggS.at[RW[b]], add=True)

    io(0, 0)

    def pipe(j, carry):
        c0 = j * 2
        io(c0 + 1, 1)
        scat(c0, 0)
        io(c0 + 2, 0)
        scat(c0 + 1, 1)
        return carry

    lax.fori_loop(0, (NCH - 1) // 2, pipe, 0)
    scat(NCH - 1, 0)
    plsc.subcore_barrier()

    def dump(j, c):
        sl = pl.ds(sid * NROWS_SUB + j * ZB, ZB)
        pltpu.sync_copy(aggS.at[sl], agg_out.at[cid].at[sl])
        return c

    lax.fori_loop(0, NROWS_SUB // ZB, dump, 0)


def _k4a(ef, rowi):
    return pl.kernel(
        _k4a_body,
        out_type=jax.ShapeDtypeStruct((NC, N_PAD, H), jnp.float32),
        mesh=_mesh(),
        compiler_params=pltpu.CompilerParams(needs_layout_passes=False),
        scratch_types=[
            pltpu.VMEM((BE,), jnp.int32),
            pltpu.VMEM((BE,), jnp.int32),
            pltpu.VMEM((BE, H), jnp.float32),
            pltpu.VMEM((BE, H), jnp.float32),
            pltpu.VMEM((ZB, D), jnp.float32),
            pltpu.VMEM_SHARED((N_PAD, H), jnp.float32),
            pltpu.SemaphoreType.DMA,
            pltpu.SemaphoreType.DMA,
        ],
    )(ef, rowi)


BE4 = 400


def _k4b_body(mflat, rowi, coli, coordf, sums_out,
              rw0, rw1, cw0, cw1, mw0, mw1, acc, pbuf, redbuf, ebuf, coordv,
              sstg, sem0, sem1):
    cid = lax.axis_index("c")
    sid = lax.axis_index("s")
    wid = sid * NC + cid
    base0 = wid * EW
    pltpu.sync_copy(coordf, coordv)

    RW = [rw0, rw1]
    CW = [cw0, cw1]
    MW = [mw0, mw1]
    SEM = [sem0, sem1]
    NCH4 = EW // BE4

    zero16 = jnp.zeros((LANES,), jnp.float32)
    ones16 = jnp.ones((LANES,), jnp.float32)

    def za_body(t, c):
        i = t // (D // LANES)
        k = lax.rem(t, D // LANES)
        acc[i, pl.ds(k * LANES, LANES)] = zero16
        return c

    lax.fori_loop(0, 320 * (D // LANES), za_body, 0)

    def ze_body(t, c):
        i = t // (D // LANES)
        k = lax.rem(t, D // LANES)
        ebuf[i, pl.ds(k * LANES, LANES)] = zero16
        return c

    lax.fori_loop(0, 32 * (D // LANES), ze_body, 0)

    def io(ci, b):
        base = base0 + ci * BE4
        pltpu.async_copy(rowi.at[pl.ds(base, BE4)], RW[b], SEM[b])
        pltpu.async_copy(coli.at[pl.ds(base, BE4)], CW[b], SEM[b])
        pltpu.async_copy(mflat.at[pl.ds(base, BE4)], MW[b], SEM[b])

    def compute(ci, b):
        base = base0 + ci * BE4
        pltpu.make_async_copy(rowi.at[pl.ds(base, BE4)], RW[b], SEM[b]).wait()
        pltpu.make_async_copy(coli.at[pl.ds(base, BE4)], CW[b], SEM[b]).wait()
        pltpu.make_async_copy(mflat.at[pl.ds(base, BE4)], MW[b], SEM[b]).wait()
        rw, cw, mw = RW[b], CW[b], MW[b]

        def grp(g, c):
            sl = pl.ds(g * LANES, LANES)
            r = rw[sl]
            r3 = r * 3
            c3 = cw[sl] * 3
            m16 = mw[sl]
            dx = _cgather(coordv, r3) - _cgather(coordv, c3)
            dy = _cgather(coordv, r3 + 1) - _cgather(coordv, c3 + 1)
            dz = _cgather(coordv, r3 + 2) - _cgather(coordv, c3 + 2)
            tx = jnp.clip(m16 * dx, -100.0, 100.0)
            ty = jnp.clip(m16 * dy, -100.0, 100.0)
            tz = jnp.clip(m16 * dz, -100.0, 100.0)
            arow = lax.shift_right_logical(r, 5)
            acol = (r & 31) * 4
            plsc.addupdate_scatter(acc, [arow, acol], tx)
            plsc.addupdate_scatter(acc, [arow, acol + 1], ty)
            plsc.addupdate_scatter(acc, [arow, acol + 2], tz)
            plsc.addupdate_scatter(acc, [arow, acol + 3], ones16)
            return c

        lax.fori_loop(0, BE4 // LANES, grp, 0)

    io(0, 0)

    def pipe(j, carry):
        c0 = j * 2
        io(c0 + 1, 1)
        compute(c0, 0)

        @pl.when(c0 + 2 < NCH4)
        def _():
            io(c0 + 2, 0)

        compute(c0 + 1, 1)
        return carry

    lax.fori_loop(0, NCH4 // 2, pipe, 0)
    compute(NCH4 - 1, 0)

    pltpu.sync_copy(acc, sstg.at[pl.ds(sid * 320, 320)])
    plsc.subcore_barrier()

    @pl.when(sid < 10)
    def _():
        def zr_body(t, c):
            i = t // (D // LANES)
            k = lax.rem(t, D // LANES)
            redbuf[i, pl.ds(k * LANES, LANES)] = zero16
            return c

        lax.fori_loop(0, 32 * (D // LANES), zr_body, 0)

        def red(p, c):
            pltpu.sync_copy(sstg.at[pl.ds(p * 320 + sid * 32, 32)], pbuf)

            def addp(t, cc):
                i = t // (D // LANES)
                k = lax.rem(t, D // LANES)
                slk = pl.ds(k * LANES, LANES)
                redbuf[i, slk] = redbuf[i, slk] + pbuf[i, slk]
                return cc

            lax.fori_loop(0, 32 * (D // LANES), addp, 0)
            return c

        lax.fori_loop(0, NS, red, 0)

        def batch(bi, c):
            def ex(g, cc):
                nloc = lax.iota(jnp.int32, LANES) + g * LANES
                srow = bi + lax.shift_right_logical(nloc, 5)
                scol = (nloc & 31) * 4
                for k in range(4):
                    val = plsc.load_gather(redbuf, [srow, scol + k])
                    plsc.store_scatter(ebuf, [nloc, jnp.full((LANES,), k, jnp.int32)], val)
                return cc

            lax.fori_loop(0, 32 // LANES, ex, 0)
            nb = sid * 1024 + bi * 32
            pltpu.sync_copy(ebuf, sums_out.at[cid].at[pl.ds(nb, 32)])
            return c

        lax.fori_loop(0, 32, batch, 0)


def _k4b(mflat, rowi, coli, coordf):
    return pl.kernel(
        _k4b_body,
        out_type=jax.ShapeDtypeStruct((NC, N_PAD, D), jnp.float32),
        mesh=_mesh(),
        compiler_params=pltpu.CompilerParams(needs_layout_passes=False),
        scratch_types=[
            pltpu.VMEM((BE4,), jnp.int32),
            pltpu.VMEM((BE4,), jnp.int32),
            pltpu.VMEM((BE4,), jnp.int32),
            pltpu.VMEM((BE4,), jnp.int32),
            pltpu.VMEM((BE4,), jnp.float32),
            pltpu.VMEM((BE4,), jnp.float32),
            pltpu.VMEM((320, D), jnp.float32),
            pltpu.VMEM((32, D), jnp.float32),
            pltpu.VMEM((32, D), jnp.float32),
            pltpu.VMEM((32, D), jnp.float32),
            pltpu.VMEM((CROWS, 128), jnp.float32),
            pltpu.VMEM_SHARED((NS * 320, D), jnp.float32),
            pltpu.SemaphoreType.DMA,
            pltpu.SemaphoreType.DMA,
        ],
    )(mflat, rowi, coli, coordf)


def _k5_body(h_ref, agg_ref, sums_ref, coord_ref, w1h_ref, w1a_ref, bn1_ref,
             wn2_ref, bn2_ref, hout_ref, cout_ref):
    agg = agg_ref[0] + agg_ref[1]
    hb = h_ref[...]
    t = jnp.maximum(
        _bdot(hb, w1h_ref[...]) + _bdot(agg, w1a_ref[...]) + bn1_ref[...], 0.0)
    hout_ref[...] = hb + _bdot(t, wn2_ref[...]) + bn2_ref[...]
    s = sums_ref[0] + sums_ref[1]
    cnt = jnp.maximum(s[:, 3:4], 1.0)
    cout_ref[...] = coord_ref[...] + s[:, 0:3] / cnt


def _k5(h, aggp, sumsp, coord, w1h, w1a, bn1, wn2, bn2):
    return pl.pallas_call(
        _k5_body,
        grid=(NNB5,),
        in_specs=[
            pl.BlockSpec((NB5, D), lambda i: (i, 0)),
            pl.BlockSpec((NC, NB5, H), lambda i: (0, i, 0)),
            pl.BlockSpec((NC, NB5, D), lambda i: (0, i, 0)),
            pl.BlockSpec((NB5, 3), lambda i: (i, 0)),
            pl.BlockSpec((D, H), lambda i: (0, 0)),
            pl.BlockSpec((H, H), lambda i: (0, 0)),
            pl.BlockSpec((1, H), lambda i: (0, 0)),
            pl.BlockSpec((H, D), lambda i: (0, 0)),
            pl.BlockSpec((1, D), lambda i: (0, 0)),
        ],
        out_specs=[
            pl.BlockSpec((NB5, D), lambda i: (i, 0)),
            pl.BlockSpec((NB5, 3), lambda i: (i, 0)),
        ],
        out_shape=[
            jax.ShapeDtypeStruct((N, D), jnp.float32),
            jax.ShapeDtypeStruct((N, 3), jnp.float32),
        ],
    )(h, aggp, sumsp, coord, w1h, w1a, bn1, wn2, bn2)


def kernel(h, edge_index, coord, edge_attr, We1, be1, We2, be2,
           Wn1, bn1, Wn2, bn2, Wc1, bc1, Wc2):
    row = edge_index[0]
    col = edge_index[1]
    wa = We1[0:D]
    wb = We1[D:2 * D]
    wr = We1[2 * D:2 * D + 1]
    wc = We1[2 * D + 1:]
    coordf = jnp.pad(coord.reshape(-1), (0, CROWS * 128 - 3 * N)).reshape(CROWS, 128)

    hwa, hwb = _k1(h, wa, wb)
    g, rad = _k2(hwa, hwb, row, col, coordf)
    rad2 = rad.reshape(NEB, 1, EB)
    ef, m2 = _k3(g, edge_attr, rad2, wc, wr, be1.reshape(1, H), We2,
                 be2.reshape(1, H), Wc1, bc1.reshape(1, H), Wc2)
    mflat = m2.reshape(-1)
    aggp = _k4a(ef, row)
    sumsp = _k4b(mflat, row, col, coordf)
    hout, cpart = _k5(h, aggp, sumsp, coord, Wn1[:D], Wn1[D:],
                      bn1.reshape(1, H), Wn2, bn2.reshape(1, D))
    return hout, cpart[:, :, None]


# speedup vs baseline: 8.3882x; 1.0559x over previous
import jax
import jax.numpy as jnp
from jax import lax
from jax.experimental import pallas as pl
from jax.experimental.pallas import tpu as pltpu
from jax.experimental.pallas import tpu_sc as plsc

N = 10000
E = 320000
D = 128
H = 128

NC = 2
NS = 16
NW = NC * NS
EW = E // NW
BE = 80
NCH = EW // BE
LANES = 16

EB = 16000
NEB = E // EB

NB = 1000
NNB = N // NB
NB5 = 1024
NNB5 = 10


CROWS = 240


def _bdot(a, b):
    return jnp.dot(a.astype(jnp.bfloat16), b.astype(jnp.bfloat16),
                   preferred_element_type=jnp.float32)


def _cgather(coordv, flat_idx):
    return plsc.load_gather(
        coordv, [lax.shift_right_logical(flat_idx, 7), flat_idx & 127])


def _mesh():
    return plsc.VectorSubcoreMesh(
        core_axis_name="c", subcore_axis_name="s", num_cores=NC, num_subcores=NS
    )


def _k1_body(h_ref, wa_ref, wb_ref, a_ref, b_ref):
    hb = h_ref[...]
    a_ref[...] = _bdot(hb, wa_ref[...])
    b_ref[...] = _bdot(hb, wb_ref[...])


def _k1(h, wa, wb):
    return pl.pallas_call(
        _k1_body,
        grid=(NNB,),
        in_specs=[
            pl.BlockSpec((NB, D), lambda i: (i, 0)),
            pl.BlockSpec((D, D), lambda i: (0, 0)),
            pl.BlockSpec((D, D), lambda i: (0, 0)),
        ],
        out_specs=[
            pl.BlockSpec((NB, D), lambda i: (i, 0)),
            pl.BlockSpec((NB, D), lambda i: (i, 0)),
        ],
        out_shape=[
            jax.ShapeDtypeStruct((N, D), jnp.float32),
            jax.ShapeDtypeStruct((N, D), jnp.float32),
        ],
    )(h, wa, wb)


def _k2_body(hwa, hwb, rowi, coli, coordf, g_out, rad_out,
             rowv0, rowv1, colv0, colv1, bufa0, bufa1, bufb0, bufb1,
             oba0, oba1, orad0, orad1, coordv,
             sema0, sema1, semb0, semb1, osem0, osem1, orsem0, orsem1):
    cid = lax.axis_index("c")
    sid = lax.axis_index("s")
    wid = sid * NC + cid
    base0 = wid * EW
    pltpu.sync_copy(coordf, coordv)

    ROWV = [rowv0, rowv1]
    COLV = [colv0, colv1]
    BUFA = [bufa0, bufa1]
    BUFB = [bufb0, bufb1]
    OBA = [oba0, oba1]
    ORAD = [orad0, orad1]
    SEMA = [sema0, sema1]
    SEMB = [semb0, semb1]
    OSEM = [osem0, osem1]
    ORSEM = [orsem0, orsem1]

    def io(ci, b):
        base = base0 + ci * BE
        pltpu.sync_copy(rowi.at[pl.ds(base, BE)], ROWV[b])
        pltpu.sync_copy(coli.at[pl.ds(base, BE)], COLV[b])
        pltpu.async_copy(hwa.at[ROWV[b]], BUFA[b], SEMA[b])
        pltpu.async_copy(hwb.at[COLV[b]], BUFB[b], SEMB[b])

    def drain_out(ci, b):
        base = base0 + ci * BE
        pltpu.make_async_copy(OBA[b], g_out.at[pl.ds(base, BE)], OSEM[b]).wait()
        pltpu.make_async_copy(ORAD[b], rad_out.at[pl.ds(base, BE)], ORSEM[b]).wait()

    def compute(ci, b):
        base = base0 + ci * BE
        pltpu.make_async_copy(hwa.at[ROWV[b]], BUFA[b], SEMA[b]).wait()
        pltpu.make_async_copy(hwb.at[COLV[b]], BUFB[b], SEMB[b]).wait()

        @pl.when(ci >= 2)
        def _():
            drain_out(ci - 2, b)

        ba, bb, oa = BUFA[b], BUFB[b], OBA[b]

        def add_body(i, c):
            for k in range(D // LANES):
                sl = pl.ds(k * LANES, LANES)
                oa[i, sl] = ba[i, sl] + bb[i, sl]
            return c

        lax.fori_loop(0, BE, add_body, 0)

        rv, cv, orv = ROWV[b], COLV[b], ORAD[b]

        def rad_body(g, c):
            sl = pl.ds(g * LANES, LANES)
            r3 = rv[sl] * 3
            c3 = cv[sl] * 3
            dx = _cgather(coordv, r3) - _cgather(coordv, c3)
            dy = _cgather(coordv, r3 + 1) - _cgather(coordv, c3 + 1)
            dz = _cgather(coordv, r3 + 2) - _cgather(coordv, c3 + 2)
            orv[sl] = dx * dx + dy * dy + dz * dz
            return c

        lax.fori_loop(0, BE // LANES, rad_body, 0)
        pltpu.async_copy(oa, g_out.at[pl.ds(base, BE)], OSEM[b])
        pltpu.async_copy(orv, rad_out.at[pl.ds(base, BE)], ORSEM[b])

    io(0, 0)

    def pipe(j, carry):
        c0 = j * 2
        io(c0 + 1, 1)
        compute(c0, 0)
        io(c0 + 2, 0)
        compute(c0 + 1, 1)
        return carry

    lax.fori_loop(0, (NCH - 1) // 2, pipe, 0)
    compute(NCH - 1, 0)
    drain_out(NCH - 2, 1)
    drain_out(NCH - 1, 0)


def _k2(hwa, hwb, rowi, coli, coordf):
    return pl.kernel(
        _k2_body,
        out_type=[
            jax.ShapeDtypeStruct((E, D), jnp.float32),
            jax.ShapeDtypeStruct((E,), jnp.float32),
        ],
        mesh=_mesh(),
        compiler_params=pltpu.CompilerParams(needs_layout_passes=False),
        scratch_types=[
            pltpu.VMEM((BE,), jnp.int32),
            pltpu.VMEM((BE,), jnp.int32),
            pltpu.VMEM((BE,), jnp.int32),
            pltpu.VMEM((BE,), jnp.int32),
            pltpu.VMEM((BE, D), jnp.float32),
            pltpu.VMEM((BE, D), jnp.float32),
            pltpu.VMEM((BE, D), jnp.float32),
            pltpu.VMEM((BE, D), jnp.float32),
            pltpu.VMEM((BE, D), jnp.float32),
            pltpu.VMEM((BE, D), jnp.float32),
            pltpu.VMEM((BE,), jnp.float32),
            pltpu.VMEM((BE,), jnp.float32),
            pltpu.VMEM((CROWS, 128), jnp.float32),
            pltpu.SemaphoreType.DMA,
            pltpu.SemaphoreType.DMA,
            pltpu.SemaphoreType.DMA,
            pltpu.SemaphoreType.DMA,
            pltpu.SemaphoreType.DMA,
            pltpu.SemaphoreType.DMA,
            pltpu.SemaphoreType.DMA,
            pltpu.SemaphoreType.DMA,
        ],
    )(hwa, hwb, rowi, coli, coordf)


def _k3_body(g_ref, ea_ref, rad_ref, wc_ref, wr_ref, be1_ref, w2_ref, be2_ref,
             wc1_ref, bc1_ref, wc2_ref, ef_ref, m_ref):
    rad_row = rad_ref[0]
    radp = lax.dot_general(rad_row, wr_ref[...], (((0,), (0,)), ((), ())),
                           preferred_element_type=jnp.float32)
    pre = (g_ref[...] + _bdot(ea_ref[...], wc_ref[...]) + radp + be1_ref[...])
    hdn = jnp.maximum(pre, 0.0)
    ef = jnp.maximum(_bdot(hdn, w2_ref[...]) + be2_ref[...], 0.0)
    ef_ref[...] = ef
    t = jnp.maximum(_bdot(ef, wc1_ref[...]) + bc1_ref[...], 0.0)
    m_ref[0] = lax.dot_general(wc2_ref[...], t, (((0,), (1,)), ((), ())),
                               preferred_element_type=jnp.float32)


def _k3(g, ea, rad2, wc, wr, be1, w2, be2, wc1, bc1, wc2):
    return pl.pallas_call(
        _k3_body,
        grid=(NEB,),
        in_specs=[
            pl.BlockSpec((EB, D), lambda i: (i, 0)),
            pl.BlockSpec((EB, H), lambda i: (i, 0)),
            pl.BlockSpec((1, 1, EB), lambda i: (i, 0, 0)),
            pl.BlockSpec((H, H), lambda i: (0, 0)),
            pl.BlockSpec((1, H), lambda i: (0, 0)),
            pl.BlockSpec((1, H), lambda i: (0, 0)),
            pl.BlockSpec((H, H), lambda i: (0, 0)),
            pl.BlockSpec((1, H), lambda i: (0, 0)),
            pl.BlockSpec((H, H), lambda i: (0, 0)),
            pl.BlockSpec((1, H), lambda i: (0, 0)),
            pl.BlockSpec((H, 1), lambda i: (0, 0)),
        ],
        out_specs=[
            pl.BlockSpec((EB, H), lambda i: (i, 0)),
            pl.BlockSpec((1, 1, EB), lambda i: (i, 0, 0)),
        ],
        out_shape=[
            jax.ShapeDtypeStruct((E, H), jnp.float32),
            jax.ShapeDtypeStruct((NEB, 1, EB), jnp.float32),
        ],
    )(g, ea, rad2, wc, wr, be1, w2, be2, wc1, bc1, wc2)


N_PAD = 10240
NROWS_SUB = N_PAD // NS
ZB = 128
NSR = N_PAD // 8


def _k4a_body(ef, rowi, agg_out, rw0, rw1, ef0, ef1, zbuf, aggS, sem0, sem1):
    cid = lax.axis_index("c")
    sid = lax.axis_index("s")
    wid = sid * NC + cid
    base0 = wid * EW

    RW = [rw0, rw1]
    EFV = [ef0, ef1]
    SEM = [sem0, sem1]

    zero16 = jnp.zeros((LANES,), jnp.float32)

    def zb_body(t, c):
        i = t // (D // LANES)
        k = lax.rem(t, D // LANES)
        zbuf[i, pl.ds(k * LANES, LANES)] = zero16
        return c

    lax.fori_loop(0, ZB * (D // LANES), zb_body, 0)

    def zc_body(j, c):
        pltpu.sync_copy(zbuf, aggS.at[pl.ds(sid * NROWS_SUB + j * ZB, ZB)])
        return c

    lax.fori_loop(0, NROWS_SUB // ZB, zc_body, 0)
    plsc.subcore_barrier()

    def io(ci, b):
        base = base0 + ci * BE
        pltpu.async_copy(rowi.at[pl.ds(base, BE)], RW[b], SEM[b])
        pltpu.async_copy(ef.at[pl.ds(base, BE)], EFV[b], SEM[b])

    def scat(ci, b):
        base = base0 + ci * BE
        pltpu.make_async_copy(rowi.at[pl.ds(base, BE)], RW[b], SEM[b]).wait()
        pltpu.make_async_copy(ef.at[pl.ds(base, BE)], EFV[b], SEM[b]).wait()
        pltpu.sync_copy(EFV[b], aggS.at[RW[b]], add=True)

    io(0, 0)

    def pipe(j, carry):
        c0 = j * 2
        io(c0 + 1, 1)
        scat(c0, 0)
        io(c0 + 2, 0)
        scat(c0 + 1, 1)
        return carry

    lax.fori_loop(0, (NCH - 1) // 2, pipe, 0)
    scat(NCH - 1, 0)
    plsc.subcore_barrier()

    def dump(j, c):
        sl = pl.ds(sid * NROWS_SUB + j * ZB, ZB)
        pltpu.sync_copy(aggS.at[sl], agg_out.at[cid].at[sl])
        return c

    lax.fori_loop(0, NROWS_SUB // ZB, dump, 0)


def _k4a(ef, rowi):
    return pl.kernel(
        _k4a_body,
        out_type=jax.ShapeDtypeStruct((NC, N_PAD, H), jnp.float32),
        mesh=_mesh(),
        compiler_params=pltpu.CompilerParams(needs_layout_passes=False),
        scratch_types=[
            pltpu.VMEM((BE,), jnp.int32),
            pltpu.VMEM((BE,), jnp.int32),
            pltpu.VMEM((BE, H), jnp.float32),
            pltpu.VMEM((BE, H), jnp.float32),
            pltpu.VMEM((ZB, D), jnp.float32),
            pltpu.VMEM_SHARED((N_PAD, H), jnp.float32),
            pltpu.SemaphoreType.DMA,
            pltpu.SemaphoreType.DMA,
        ],
    )(ef, rowi)


BE4 = 400


def _k4b_body(mflat, rowi, coli, coordf, sums_out,
              rw0, rw1, cw0, cw1, mw0, mw1, acc, pbuf, redbuf, ebuf, coordv,
              sstg, sem0, sem1):
    cid = lax.axis_index("c")
    sid = lax.axis_index("s")
    wid = sid * NC + cid
    base0 = wid * EW
    pltpu.sync_copy(coordf, coordv)

    RW = [rw0, rw1]
    CW = [cw0, cw1]
    MW = [mw0, mw1]
    SEM = [sem0, sem1]
    NCH4 = EW // BE4

    zero16 = jnp.zeros((LANES,), jnp.float32)
    ones16 = jnp.ones((LANES,), jnp.float32)

    def za_body(t, c):
        i = t // (D // LANES)
        k = lax.rem(t, D // LANES)
        acc[i, pl.ds(k * LANES, LANES)] = zero16
        return c

    lax.fori_loop(0, 320 * (D // LANES), za_body, 0)

    def ze_body(t, c):
        i = t // (D // LANES)
        k = lax.rem(t, D // LANES)
        ebuf[i, pl.ds(k * LANES, LANES)] = zero16
        return c

    lax.fori_loop(0, 32 * (D // LANES), ze_body, 0)

    def io(ci, b):
        base = base0 + ci * BE4
        pltpu.async_copy(rowi.at[pl.ds(base, BE4)], RW[b], SEM[b])
        pltpu.async_copy(coli.at[pl.ds(base, BE4)], CW[b], SEM[b])
        pltpu.async_copy(mflat.at[pl.ds(base, BE4)], MW[b], SEM[b])

    def compute(ci, b):
        base = base0 + ci * BE4
        pltpu.make_async_copy(rowi.at[pl.ds(base, BE4)], RW[b], SEM[b]).wait()
        pltpu.make_async_copy(coli.at[pl.ds(base, BE4)], CW[b], SEM[b]).wait()
        pltpu.make_async_copy(mflat.at[pl.ds(base, BE4)], MW[b], SEM[b]).wait()
        rw, cw, mw = RW[b], CW[b], MW[b]

        def grp(g, c):
            sl = pl.ds(g * LANES, LANES)
            r = rw[sl]
            r3 = r * 3
            c3 = cw[sl] * 3
            m16 = mw[sl]
            dx = _cgather(coordv, r3) - _cgather(coordv, c3)
            dy = _cgather(coordv, r3 + 1) - _cgather(coordv, c3 + 1)
            dz = _cgather(coordv, r3 + 2) - _cgather(coordv, c3 + 2)
            tx = jnp.clip(m16 * dx, -100.0, 100.0)
            ty = jnp.clip(m16 * dy, -100.0, 100.0)
            tz = jnp.clip(m16 * dz, -100.0, 100.0)
            arow = lax.shift_right_logical(r, 5)
            acol = (r & 31) * 4
            plsc.addupdate_scatter(acc, [arow, acol], tx)
            plsc.addupdate_scatter(acc, [arow, acol + 1], ty)
            plsc.addupdate_scatter(acc, [arow, acol + 2], tz)
            plsc.addupdate_scatter(acc, [arow, acol + 3], ones16)
            return c

        lax.fori_loop(0, BE4 // LANES, grp, 0)

    io(0, 0)

    def pipe(j, carry):
        c0 = j * 2
        io(c0 + 1, 1)
        compute(c0, 0)

        @pl.when(c0 + 2 < NCH4)
        def _():
            io(c0 + 2, 0)

        compute(c0 + 1, 1)
        return carry

    lax.fori_loop(0, NCH4 // 2, pipe, 0)
    compute(NCH4 - 1, 0)

    pltpu.sync_copy(acc, sstg.at[pl.ds(sid * 320, 320)])
    plsc.subcore_barrier()

    @pl.when(sid < 10)
    def _():
        def zr_body(t, c):
            i = t // (D // LANES)
            k = lax.rem(t, D // LANES)
            redbuf[i, pl.ds(k * LANES, LANES)] = zero16
            return c

        lax.fori_loop(0, 32 * (D // LANES), zr_body, 0)

        def red(p, c):
            pltpu.sync_copy(sstg.at[pl.ds(p * 320 + sid * 32, 32)], pbuf)

            def addp(t, cc):
                i = t // (D // LANES)
                k = lax.rem(t, D // LANES)
                slk = pl.ds(k * LANES, LANES)
                redbuf[i, slk] = redbuf[i, slk] + pbuf[i, slk]
                return cc

            lax.fori_loop(0, 32 * (D // LANES), addp, 0)
            return c

        lax.fori_loop(0, NS, red, 0)

        def batch(bi, c):
            def ex(g, cc):
                nloc = lax.iota(jnp.int32, LANES) + g * LANES
                srow = bi + lax.shift_right_logical(nloc, 5)
                scol = (nloc & 31) * 4
                for k in range(4):
                    val = plsc.load_gather(redbuf, [srow, scol + k])
                    plsc.store_scatter(ebuf, [nloc, jnp.full((LANES,), k, jnp.int32)], val)
                return cc

            lax.fori_loop(0, 32 // LANES, ex, 0)
            nb = sid * 1024 + bi * 32
            pltpu.sync_copy(ebuf, sums_out.at[cid].at[pl.ds(nb, 32)])
            return c

        lax.fori_loop(0, 32, batch, 0)


def _k4b(mflat, rowi, coli, coordf):
    return pl.kernel(
        _k4b_body,
        out_type=jax.ShapeDtypeStruct((NC, N_PAD, D), jnp.float32),
        mesh=_mesh(),
        compiler_params=pltpu.CompilerParams(needs_layout_passes=False),
        scratch_types=[
            pltpu.VMEM((BE4,), jnp.int32),
            pltpu.VMEM((BE4,), jnp.int32),
            pltpu.VMEM((BE4,), jnp.int32),
            pltpu.VMEM((BE4,), jnp.int32),
            pltpu.VMEM((BE4,), jnp.float32),
            pltpu.VMEM((BE4,), jnp.float32),
            pltpu.VMEM((320, D), jnp.float32),
            pltpu.VMEM((32, D), jnp.float32),
            pltpu.VMEM((32, D), jnp.float32),
            pltpu.VMEM((32, D), jnp.float32),
            pltpu.VMEM((CROWS, 128), jnp.float32),
            pltpu.VMEM_SHARED((NS * 320, D), jnp.float32),
            pltpu.SemaphoreType.DMA,
            pltpu.SemaphoreType.DMA,
        ],
    )(mflat, rowi, coli, coordf)


def _k5_body(h_ref, agg_ref, sums_ref, coord_ref, w1h_ref, w1a_ref, bn1_ref,
             wn2_ref, bn2_ref, hout_ref, cout_ref):
    agg = agg_ref[0] + agg_ref[1]
    hb = h_ref[...]
    t = jnp.maximum(
        _bdot(hb, w1h_ref[...]) + _bdot(agg, w1a_ref[...]) + bn1_ref[...], 0.0)
    hout_ref[...] = hb + _bdot(t, wn2_ref[...]) + bn2_ref[...]
    s = sums_ref[0] + sums_ref[1]
    cnt = jnp.maximum(s[:, 3:4], 1.0)
    cout_ref[...] = coord_ref[...] + s[:, 0:3] / cnt


def _k5(h, aggp, sumsp, coord, w1h, w1a, bn1, wn2, bn2):
    return pl.pallas_call(
        _k5_body,
        grid=(NNB5,),
        in_specs=[
            pl.BlockSpec((NB5, D), lambda i: (i, 0)),
            pl.BlockSpec((NC, NB5, H), lambda i: (0, i, 0)),
            pl.BlockSpec((NC, NB5, D), lambda i: (0, i, 0)),
            pl.BlockSpec((NB5, 3), lambda i: (i, 0)),
            pl.BlockSpec((D, H), lambda i: (0, 0)),
            pl.BlockSpec((H, H), lambda i: (0, 0)),
            pl.BlockSpec((1, H), lambda i: (0, 0)),
            pl.BlockSpec((H, D), lambda i: (0, 0)),
            pl.BlockSpec((1, D), lambda i: (0, 0)),
        ],
        out_specs=[
            pl.BlockSpec((NB5, D), lambda i: (i, 0)),
            pl.BlockSpec((NB5, 3), lambda i: (i, 0)),
        ],
        out_shape=[
            jax.ShapeDtypeStruct((N, D), jnp.float32),
            jax.ShapeDtypeStruct((N, 3), jnp.float32),
        ],
    )(h, aggp, sumsp, coord, w1h, w1a, bn1, wn2, bn2)


def kernel(h, edge_index, coord, edge_attr, We1, be1, We2, be2,
           Wn1, bn1, Wn2, bn2, Wc1, bc1, Wc2):
    row = edge_index[0]
    col = edge_index[1]
    wa = We1[0:D]
    wb = We1[D:2 * D]
    wr = We1[2 * D:2 * D + 1]
    wc = We1[2 * D + 1:]
    coordf = jnp.pad(coord.reshape(-1), (0, CROWS * 128 - 3 * N)).reshape(CROWS, 128)

    hwa, hwb = _k1(h, wa, wb)
    g, rad = _k2(hwa, hwb, row, col, coordf)
    rad2 = rad.reshape(NEB, 1, EB)
    ef, m2 = _k3(g, edge_attr, rad2, wc, wr, be1.reshape(1, H), We2,
                 be2.reshape(1, H), Wc1, bc1.reshape(1, H), Wc2)
    mflat = m2.reshape(-1)
    aggp = _k4a(ef, row)
    sumsp = _k4b(mflat, row, col, coordf)
    hout, cpart = _k5(h, aggp, sumsp, coord, Wn1[:D], Wn1[D:],
                      bn1.reshape(1, H), Wn2, bn2.reshape(1, D))
    return hout, cpart[:, :, None]
